# Initial kernel scaffold; baseline (speedup 1.0000x reference)
import jax
import jax.numpy as jnp
from jax.experimental import pallas as pl


def kernel(x, edge_index, edge_attr, batch, params):
    raise NotImplementedError("write your pallas kernel here")


# speedup vs baseline: 1.5837x; 1.5837x over previous
import functools

import jax
import jax.numpy as jnp
from jax import lax
from jax.experimental import pallas as pl
from jax.experimental.pallas import tpu as pltpu
from jax.experimental.pallas import tpu_sc as plsc

N_NODES = 50000
N_EDGES = 800000
HIDDEN = 64
N_GRAPHS = 512

F32 = jnp.float32


def _linear_body(x_ref, w_ref, b_ref, o_ref, *, relu):
    y = jnp.dot(x_ref[...], w_ref[...], preferred_element_type=F32) + b_ref[...]
    if relu:
        y = jnp.maximum(y, 0.0)
    o_ref[...] = y


def _linear(x, w, b, *, relu=False, block_rows):
    m, k = x.shape
    n = w.shape[1]
    assert m % block_rows == 0
    return pl.pallas_call(
        functools.partial(_linear_body, relu=relu),
        grid=(m // block_rows,),
        in_specs=[
            pl.BlockSpec((block_rows, k), lambda i: (i, 0)),
            pl.BlockSpec((k, n), lambda i: (0, 0)),
            pl.BlockSpec((1, n), lambda i: (0, 0)),
        ],
        out_specs=pl.BlockSpec((block_rows, n), lambda i: (i, 0)),
        out_shape=jax.ShapeDtypeStruct((m, n), F32),
    )(x, w, b.reshape(1, n))


def _mlp_stats_body(x_ref, a_ref, w1_ref, b1_ref, w2_ref, b2_ref,
                    z_ref, s_ref, ss_ref):
    h = x_ref[...] + a_ref[...]
    h1 = jnp.dot(h, w1_ref[...], preferred_element_type=F32) + b1_ref[...]
    h1 = jnp.maximum(h1, 0.0)
    z = jnp.dot(h1, w2_ref[...], preferred_element_type=F32) + b2_ref[...]
    z_ref[...] = z

    @pl.when(pl.program_id(0) == 0)
    def _zero():
        s_ref[...] = jnp.zeros_like(s_ref)
        ss_ref[...] = jnp.zeros_like(ss_ref)

    s_ref[...] += jnp.sum(z, axis=0, keepdims=True)
    ss_ref[...] += jnp.sum(z * z, axis=0, keepdims=True)


def _mlp_stats(x, aggr, p, *, block_rows=2000):
    m, d = x.shape
    assert m % block_rows == 0
    w1, b1 = p["l1"]["w"], p["l1"]["b"].reshape(1, d)
    w2, b2 = p["l2"]["w"], p["l2"]["b"].reshape(1, d)
    z, s, ss = pl.pallas_call(
        _mlp_stats_body,
        grid=(m // block_rows,),
        in_specs=[
            pl.BlockSpec((block_rows, d), lambda i: (i, 0)),
            pl.BlockSpec((block_rows, d), lambda i: (i, 0)),
            pl.BlockSpec((d, d), lambda i: (0, 0)),
            pl.BlockSpec((1, d), lambda i: (0, 0)),
            pl.BlockSpec((d, d), lambda i: (0, 0)),
            pl.BlockSpec((1, d), lambda i: (0, 0)),
        ],
        out_specs=[
            pl.BlockSpec((block_rows, d), lambda i: (i, 0)),
            pl.BlockSpec((1, d), lambda i: (0, 0)),
            pl.BlockSpec((1, d), lambda i: (0, 0)),
        ],
        out_shape=[
            jax.ShapeDtypeStruct((m, d), F32),
            jax.ShapeDtypeStruct((1, d), F32),
            jax.ShapeDtypeStruct((1, d), F32),
        ],
    )(x, aggr, w1, b1, w2, b2)
    return z, s, ss


def _bn_apply_body(z_ref, s_ref, ss_ref, g_ref, b_ref, o_ref, *, m):
    mean = s_ref[...] / m
    var = ss_ref[...] / m - mean * mean
    inv = jax.lax.rsqrt(var + 1e-5)
    o_ref[...] = jnp.maximum((z_ref[...] - mean) * inv * g_ref[...] + b_ref[...], 0.0)


def _bn_apply(z, s, ss, p, *, block_rows=2000):
    m, d = z.shape
    return pl.pallas_call(
        functools.partial(_bn_apply_body, m=float(m)),
        grid=(m // block_rows,),
        in_specs=[
            pl.BlockSpec((block_rows, d), lambda i: (i, 0)),
            pl.BlockSpec((1, d), lambda i: (0, 0)),
            pl.BlockSpec((1, d), lambda i: (0, 0)),
            pl.BlockSpec((1, d), lambda i: (0, 0)),
            pl.BlockSpec((1, d), lambda i: (0, 0)),
        ],
        out_specs=pl.BlockSpec((block_rows, d), lambda i: (i, 0)),
        out_shape=jax.ShapeDtypeStruct((m, d), F32),
    )(z, s, ss, p["g"].reshape(1, d), p["b"].reshape(1, d))


def _pool_body(xf_ref, xb_ref, batch_ref, o_ref):
    seg = batch_ref[...].reshape(1, -1).astype(jnp.int32)
    gids = jax.lax.broadcasted_iota(jnp.int32, (N_GRAPHS, seg.shape[1]), 0)
    onehot = (gids == seg).astype(F32)
    h = jnp.concatenate([xf_ref[...], xb_ref[...]], axis=1)

    @pl.when(pl.program_id(0) == 0)
    def _zero():
        o_ref[...] = jnp.zeros_like(o_ref)

    o_ref[...] += jnp.dot(onehot, h, preferred_element_type=F32)


def _pool(xf, xb, batch, *, block_rows=2000):
    m, d = xf.shape
    batch3 = batch.astype(jnp.int32).reshape(m // block_rows, 1, block_rows)
    return pl.pallas_call(
        _pool_body,
        grid=(m // block_rows,),
        in_specs=[
            pl.BlockSpec((block_rows, d), lambda i: (i, 0)),
            pl.BlockSpec((block_rows, d), lambda i: (i, 0)),
            pl.BlockSpec((1, 1, block_rows), lambda i: (i, 0, 0)),
        ],
        out_specs=pl.BlockSpec((N_GRAPHS, 2 * d), lambda i: (0, 0)),
        out_shape=jax.ShapeDtypeStruct((N_GRAPHS, 2 * d), F32),
    )(xf, xb, batch3)


def _head_body(x_ref, w1_ref, b1_ref, w2_ref, b2_ref, o_ref):
    h = jnp.dot(x_ref[...], w1_ref[...], preferred_element_type=F32) + b1_ref[...]
    h = jnp.maximum(h, 0.0)
    o_ref[...] = jnp.dot(h, w2_ref[...], preferred_element_type=F32) + b2_ref[...]


def _head(pooled, p1, p2):
    m, k = pooled.shape
    d = p1["w"].shape[1]
    out = pl.pallas_call(
        _head_body,
        in_specs=[
            pl.BlockSpec((m, k), lambda: (0, 0)),
            pl.BlockSpec((k, d), lambda: (0, 0)),
            pl.BlockSpec((1, d), lambda: (0, 0)),
            pl.BlockSpec((d, 1), lambda: (0, 0)),
            pl.BlockSpec((1, 1), lambda: (0, 0)),
        ],
        out_specs=pl.BlockSpec((m, 1), lambda: (0, 0)),
        out_shape=jax.ShapeDtypeStruct((m, 1), F32),
    )(pooled, p1["w"], p1["b"].reshape(1, d), p2["w"], p2["b"].reshape(1, 1))
    return out[:, 0]


_HALF = N_NODES // 2
_RPW = 1568
_SPM_ROWS = 16 * _RPW
_DUMP = _HALF
_EC = 80
_EPS = N_EDGES // 16
_NCH = _EPS // _EC
_ZR = _RPW // 8


def _edge_aggr(t, ea, jj, ii):
    mesh = plsc.VectorSubcoreMesh(core_axis_name="c", subcore_axis_name="s")

    @functools.partial(
        pl.kernel,
        mesh=mesh,
        out_type=jax.ShapeDtypeStruct((N_NODES, HIDDEN), F32),
        compiler_params=pltpu.CompilerParams(use_tc_tiling_on_sc=False),
        scratch_types=[
            pltpu.VMEM((_EC,), jnp.int32),
            pltpu.VMEM((_EC,), jnp.int32),
            pltpu.VMEM((_EC,), jnp.int32),
            pltpu.VMEM((_EC, HIDDEN), F32),
            pltpu.VMEM((_EC, HIDDEN), F32),
            pltpu.VMEM((_ZR, HIDDEN), F32),
            pltpu.VMEM_SHARED((_SPM_ROWS, HIDDEN), F32),
        ],
    )
    def k(t_hbm, ea_hbm, j_hbm, i_hbm, out_hbm,
          jbuf, ibuf, ilbuf, xjbuf, eabuf, zbuf, spm):
        c = lax.axis_index("c")
        s = lax.axis_index("s")

        zv = jnp.zeros((16,), F32)

        @pl.loop(0, _ZR)
        def _zrow(r):
            for q in range(HIDDEN // 16):
                zbuf.at[r, pl.ds(q * 16, 16)][...] = zv

        @pl.loop(0, 8)
        def _zcopy(b):
            pltpu.sync_copy(zbuf, spm.at[pl.ds(s * _RPW + b * _ZR, _ZR)])

        plsc.subcore_barrier()

        lo = c * _HALF

        @pl.loop(0, _NCH)
        def _chunk(kk):
            base = s * _EPS + kk * _EC
            pltpu.sync_copy(j_hbm.at[pl.ds(base, _EC)], jbuf)
            pltpu.sync_copy(i_hbm.at[pl.ds(base, _EC)], ibuf)
            pltpu.sync_copy(ea_hbm.at[pl.ds(base, _EC)], eabuf)
            pltpu.sync_copy(t_hbm.at[jbuf], xjbuf)

            @pl.loop(0, _EC, step=16)
            def _locidx(q):
                iv = ibuf.at[pl.ds(q, 16)][...]
                ilv = iv - lo
                valid = (ilv >= 0) & (ilv < _HALF)
                dump = _DUMP + (iv & 63)
                ilbuf.at[pl.ds(q, 16)][...] = jnp.where(valid, ilv, dump)

            @pl.loop(0, _EC)
            def _msg(r):
                for q in range(HIDDEN // 16):
                    sl = pl.ds(q * 16, 16)
                    v = xjbuf.at[r, sl][...] + eabuf.at[r, sl][...]
                    xjbuf.at[r, sl][...] = jnp.maximum(v, 0.0)

            pltpu.sync_copy(xjbuf, spm.at[ilbuf], add=True)

        plsc.subcore_barrier()

        start = jnp.minimum(s * _RPW, _HALF - _RPW)
        pltpu.sync_copy(spm.at[pl.ds(start, _RPW)],
                        out_hbm.at[pl.ds(lo + start, _RPW)])

    return k(t, ea, jj, ii)


def _conv(t, ea, jj, ii, pmlp, pbn):
    aggr = _edge_aggr(t, ea, jj, ii)
    z, s, ss = _mlp_stats(t, aggr, pmlp)
    return _bn_apply(z, s, ss, pbn)


def kernel(x, edge_index, edge_attr, batch, params):
    p = params
    h = _linear(x, p["node_enc"]["w"], p["node_enc"]["b"], block_rows=2000)
    ea = _linear(edge_attr, p["edge_enc"]["w"], p["edge_enc"]["b"], block_rows=8000)
    src = edge_index[0]
    dst = edge_index[1]

    xf = _conv(h, ea, src, dst, p["f_conv1"], p["f_bn1"])
    xf = _conv(xf, ea, src, dst, p["f_conv2"], p["f_bn2"])
    xb = _conv(h, ea, dst, src, p["b_conv1"], p["b_bn1"])
    xb = _conv(xb, ea, dst, src, p["b_conv2"], p["b_bn2"])

    pooled = _pool(xf, xb, batch)
    return _head(pooled, p["head1"], p["head2"])


# speedup vs baseline: 4.0800x; 2.5762x over previous
import functools

import jax
import jax.numpy as jnp
from jax import lax
from jax.experimental import pallas as pl
from jax.experimental.pallas import tpu as pltpu
from jax.experimental.pallas import tpu_sc as plsc

N_NODES = 50000
N_EDGES = 800000
HIDDEN = 64
N_GRAPHS = 512

F32 = jnp.float32


def _linear_body(x_ref, w_ref, b_ref, o_ref, *, relu):
    y = jnp.dot(x_ref[...], w_ref[...], preferred_element_type=F32) + b_ref[...]
    if relu:
        y = jnp.maximum(y, 0.0)
    o_ref[...] = y


def _linear(x, w, b, *, relu=False, block_rows):
    m, k = x.shape
    n = w.shape[1]
    assert m % block_rows == 0
    return pl.pallas_call(
        functools.partial(_linear_body, relu=relu),
        grid=(m // block_rows,),
        in_specs=[
            pl.BlockSpec((block_rows, k), lambda i: (i, 0)),
            pl.BlockSpec((k, n), lambda i: (0, 0)),
            pl.BlockSpec((1, n), lambda i: (0, 0)),
        ],
        out_specs=pl.BlockSpec((block_rows, n), lambda i: (i, 0)),
        out_shape=jax.ShapeDtypeStruct((m, n), F32),
    )(x, w, b.reshape(1, n))


def _mlp_stats_body(x_ref, a_ref, w1_ref, b1_ref, w2_ref, b2_ref,
                    z_ref, s_ref, ss_ref):
    h = x_ref[...] + a_ref[...]
    h1 = jnp.dot(h, w1_ref[...], preferred_element_type=F32) + b1_ref[...]
    h1 = jnp.maximum(h1, 0.0)
    z = jnp.dot(h1, w2_ref[...], preferred_element_type=F32) + b2_ref[...]
    z_ref[...] = z

    @pl.when(pl.program_id(0) == 0)
    def _zero():
        s_ref[...] = jnp.zeros_like(s_ref)
        ss_ref[...] = jnp.zeros_like(ss_ref)

    s_ref[...] += jnp.sum(z, axis=0, keepdims=True)
    ss_ref[...] += jnp.sum(z * z, axis=0, keepdims=True)


def _mlp_stats(x, aggr, p, *, block_rows=2000):
    m, d = x.shape
    assert m % block_rows == 0
    w1, b1 = p["l1"]["w"], p["l1"]["b"].reshape(1, d)
    w2, b2 = p["l2"]["w"], p["l2"]["b"].reshape(1, d)
    z, s, ss = pl.pallas_call(
        _mlp_stats_body,
        grid=(m // block_rows,),
        in_specs=[
            pl.BlockSpec((block_rows, d), lambda i: (i, 0)),
            pl.BlockSpec((block_rows, d), lambda i: (i, 0)),
            pl.BlockSpec((d, d), lambda i: (0, 0)),
            pl.BlockSpec((1, d), lambda i: (0, 0)),
            pl.BlockSpec((d, d), lambda i: (0, 0)),
            pl.BlockSpec((1, d), lambda i: (0, 0)),
        ],
        out_specs=[
            pl.BlockSpec((block_rows, d), lambda i: (i, 0)),
            pl.BlockSpec((1, d), lambda i: (0, 0)),
            pl.BlockSpec((1, d), lambda i: (0, 0)),
        ],
        out_shape=[
            jax.ShapeDtypeStruct((m, d), F32),
            jax.ShapeDtypeStruct((1, d), F32),
            jax.ShapeDtypeStruct((1, d), F32),
        ],
    )(x, aggr, w1, b1, w2, b2)
    return z, s, ss


def _bn_apply_body(z_ref, s_ref, ss_ref, g_ref, b_ref, o_ref, *, m):
    mean = s_ref[...] / m
    var = ss_ref[...] / m - mean * mean
    inv = jax.lax.rsqrt(var + 1e-5)
    o_ref[...] = jnp.maximum((z_ref[...] - mean) * inv * g_ref[...] + b_ref[...], 0.0)


def _bn_apply(z, s, ss, p, *, block_rows=2000):
    m, d = z.shape
    return pl.pallas_call(
        functools.partial(_bn_apply_body, m=float(m)),
        grid=(m // block_rows,),
        in_specs=[
            pl.BlockSpec((block_rows, d), lambda i: (i, 0)),
            pl.BlockSpec((1, d), lambda i: (0, 0)),
            pl.BlockSpec((1, d), lambda i: (0, 0)),
            pl.BlockSpec((1, d), lambda i: (0, 0)),
            pl.BlockSpec((1, d), lambda i: (0, 0)),
        ],
        out_specs=pl.BlockSpec((block_rows, d), lambda i: (i, 0)),
        out_shape=jax.ShapeDtypeStruct((m, d), F32),
    )(z, s, ss, p["g"].reshape(1, d), p["b"].reshape(1, d))


def _pool_body(xf_ref, xb_ref, batch_ref, o_ref):
    seg = batch_ref[...].reshape(1, -1).astype(jnp.int32)
    gids = jax.lax.broadcasted_iota(jnp.int32, (N_GRAPHS, seg.shape[1]), 0)
    onehot = (gids == seg).astype(F32)
    h = jnp.concatenate([xf_ref[...], xb_ref[...]], axis=1)

    @pl.when(pl.program_id(0) == 0)
    def _zero():
        o_ref[...] = jnp.zeros_like(o_ref)

    o_ref[...] += jnp.dot(onehot, h, preferred_element_type=F32)


def _pool(xf, xb, batch, *, block_rows=2000):
    m, d = xf.shape
    batch3 = batch.astype(jnp.int32).reshape(m // block_rows, 1, block_rows)
    return pl.pallas_call(
        _pool_body,
        grid=(m // block_rows,),
        in_specs=[
            pl.BlockSpec((block_rows, d), lambda i: (i, 0)),
            pl.BlockSpec((block_rows, d), lambda i: (i, 0)),
            pl.BlockSpec((1, 1, block_rows), lambda i: (i, 0, 0)),
        ],
        out_specs=pl.BlockSpec((N_GRAPHS, 2 * d), lambda i: (0, 0)),
        out_shape=jax.ShapeDtypeStruct((N_GRAPHS, 2 * d), F32),
    )(xf, xb, batch3)


def _head_body(x_ref, w1_ref, b1_ref, w2_ref, b2_ref, o_ref):
    h = jnp.dot(x_ref[...], w1_ref[...], preferred_element_type=F32) + b1_ref[...]
    h = jnp.maximum(h, 0.0)
    o_ref[...] = jnp.dot(h, w2_ref[...], preferred_element_type=F32) + b2_ref[...]


def _head(pooled, p1, p2):
    m, k = pooled.shape
    d = p1["w"].shape[1]
    out = pl.pallas_call(
        _head_body,
        in_specs=[
            pl.BlockSpec((m, k), lambda: (0, 0)),
            pl.BlockSpec((k, d), lambda: (0, 0)),
            pl.BlockSpec((1, d), lambda: (0, 0)),
            pl.BlockSpec((d, 1), lambda: (0, 0)),
            pl.BlockSpec((1, 1), lambda: (0, 0)),
        ],
        out_specs=pl.BlockSpec((m, 1), lambda: (0, 0)),
        out_shape=jax.ShapeDtypeStruct((m, 1), F32),
    )(pooled, p1["w"], p1["b"].reshape(1, d), p2["w"], p2["b"].reshape(1, 1))
    return out[:, 0]


_HALF = N_NODES // 2
_RPW = 1568
_SPM_ROWS = 16 * _RPW
_DUMP = _HALF
_EC = 80
_EPS = N_EDGES // 16
_NCH = _EPS // _EC
_ZR = _RPW // 32


_NBUF = 2


def _edge_aggr(t, ea, jj, ii):
    mesh = plsc.VectorSubcoreMesh(core_axis_name="c", subcore_axis_name="s")

    scratch = []
    for _ in range(_NBUF):
        scratch += [
            pltpu.VMEM((_EC,), jnp.int32),
            pltpu.VMEM((_EC,), jnp.int32),
            pltpu.VMEM((_EC,), jnp.int32),
            pltpu.VMEM((_EC, HIDDEN), F32),
            pltpu.VMEM((_EC, HIDDEN), F32),
        ]
    scratch += [
        pltpu.VMEM((_ZR, HIDDEN), F32),
        pltpu.VMEM_SHARED((_SPM_ROWS, HIDDEN), F32),
    ]
    scratch += [pltpu.SemaphoreType.DMA] * (5 * _NBUF)

    @functools.partial(
        pl.kernel,
        mesh=mesh,
        out_type=jax.ShapeDtypeStruct((N_NODES, HIDDEN), F32),
        compiler_params=pltpu.CompilerParams(use_tc_tiling_on_sc=False),
        scratch_types=scratch,
    )
    def k(t_hbm, ea_hbm, j_hbm, i_hbm, out_hbm, *refs):
        bufs = [refs[5 * b:5 * b + 5] for b in range(_NBUF)]
        zbuf = refs[5 * _NBUF]
        spm = refs[5 * _NBUF + 1]
        sems = refs[5 * _NBUF + 2:]
        sj = sems[0:_NBUF]
        si = sems[_NBUF:2 * _NBUF]
        sea = sems[2 * _NBUF:3 * _NBUF]
        sg = sems[3 * _NBUF:4 * _NBUF]
        ssc = sems[4 * _NBUF:5 * _NBUF]

        c = lax.axis_index("c")
        s = lax.axis_index("s")
        lo = c * _HALF
        ebase = s * _EPS
        n_iter = _NCH // _NBUF

        zv = jnp.zeros((16,), F32)

        @pl.loop(0, _ZR)
        def _zrow(r):
            for q in range(HIDDEN // 16):
                zbuf.at[r, pl.ds(q * 16, 16)][...] = zv

        @pl.loop(0, 32)
        def _zcopy(b):
            pltpu.sync_copy(zbuf, spm.at[pl.ds(s * _RPW + b * _ZR, _ZR)])

        plsc.subcore_barrier()

        def prefetch(b, kk):
            base = ebase + kk * _EC
            jb, ib, ilb, xjb, eab = bufs[b]
            pltpu.async_copy(j_hbm.at[pl.ds(base, _EC)], jb, sj[b])
            pltpu.async_copy(i_hbm.at[pl.ds(base, _EC)], ib, si[b])
            pltpu.async_copy(ea_hbm.at[pl.ds(base, _EC)], eab, sea[b])

        for b in range(_NBUF):
            prefetch(b, b)

        @pl.loop(0, n_iter)
        def _iter(k2):
            for b in range(_NBUF):
                jb, ib, ilb, xjb, eab = bufs[b]

                @pl.when(k2 > 0)
                def _drain():
                    pltpu.make_async_copy(xjb, spm.at[ilb], ssc[b]).wait()

                pltpu.make_async_copy(j_hbm.at[pl.ds(ebase, _EC)], jb,
                                      sj[b]).wait()
                pltpu.async_copy(t_hbm.at[jb], xjb, sg[b])

            for b in range(_NBUF):
                jb, ib, ilb, xjb, eab = bufs[b]
                kk = k2 * _NBUF + b
                base = ebase + kk * _EC

                pltpu.make_async_copy(i_hbm.at[pl.ds(ebase, _EC)], ib,
                                      si[b]).wait()

                @pl.loop(0, _EC, step=16)
                def _locidx(q):
                    iv = ib.at[pl.ds(q, 16)][...]
                    ilv = iv - lo
                    valid = (ilv >= 0) & (ilv < _HALF)
                    dump = _DUMP + (iv & 63)
                    ilb.at[pl.ds(q, 16)][...] = jnp.where(valid, ilv, dump)

                pltpu.make_async_copy(ea_hbm.at[pl.ds(base, _EC)], eab,
                                      sea[b]).wait()
                pltpu.make_async_copy(t_hbm.at[jb], xjb, sg[b]).wait()

                @pl.loop(0, _EC)
                def _msg(r):
                    for q in range(HIDDEN // 16):
                        sl = pl.ds(q * 16, 16)
                        v = xjb.at[r, sl][...] + eab.at[r, sl][...]
                        xjb.at[r, sl][...] = jnp.maximum(v, 0.0)

                pltpu.async_copy(xjb, spm.at[ilb], ssc[b], add=True)

                @pl.when(kk + _NBUF < n_iter * _NBUF)
                def _next():
                    prefetch(b, kk + _NBUF)

        for b in range(_NBUF):
            jb, ib, ilb, xjb, eab = bufs[b]
            pltpu.make_async_copy(xjb, spm.at[ilb], ssc[b]).wait()

        for kk in range(n_iter * _NBUF, _NCH):
            jb, ib, ilb, xjb, eab = bufs[0]
            base = ebase + kk * _EC
            pltpu.sync_copy(j_hbm.at[pl.ds(base, _EC)], jb)
            pltpu.sync_copy(i_hbm.at[pl.ds(base, _EC)], ib)
            pltpu.sync_copy(ea_hbm.at[pl.ds(base, _EC)], eab)
            pltpu.sync_copy(t_hbm.at[jb], xjb)

            @pl.loop(0, _EC, step=16)
            def _locidx_t(q):
                iv = ib.at[pl.ds(q, 16)][...]
                ilv = iv - lo
                valid = (ilv >= 0) & (ilv < _HALF)
                dump = _DUMP + (iv & 63)
                ilb.at[pl.ds(q, 16)][...] = jnp.where(valid, ilv, dump)

            @pl.loop(0, _EC)
            def _msg_t(r):
                for q in range(HIDDEN // 16):
                    sl = pl.ds(q * 16, 16)
                    v = xjb.at[r, sl][...] + eab.at[r, sl][...]
                    xjb.at[r, sl][...] = jnp.maximum(v, 0.0)

            pltpu.sync_copy(xjb, spm.at[ilb], add=True)

        plsc.subcore_barrier()

        start = jnp.minimum(s * _RPW, _HALF - _RPW)
        pltpu.sync_copy(spm.at[pl.ds(start, _RPW)],
                        out_hbm.at[pl.ds(lo + start, _RPW)])

    return k(t, ea, jj, ii)


def _conv(t, ea, jj, ii, pmlp, pbn):
    aggr = _edge_aggr(t, ea, jj, ii)
    z, s, ss = _mlp_stats(t, aggr, pmlp)
    return _bn_apply(z, s, ss, pbn)


def kernel(x, edge_index, edge_attr, batch, params):
    p = params
    h = _linear(x, p["node_enc"]["w"], p["node_enc"]["b"], block_rows=2000)
    ea = _linear(edge_attr, p["edge_enc"]["w"], p["edge_enc"]["b"], block_rows=8000)
    src = edge_index[0]
    dst = edge_index[1]

    xf = _conv(h, ea, src, dst, p["f_conv1"], p["f_bn1"])
    xf = _conv(xf, ea, src, dst, p["f_conv2"], p["f_bn2"])
    xb = _conv(h, ea, dst, src, p["b_conv1"], p["b_bn1"])
    xb = _conv(xb, ea, dst, src, p["b_conv2"], p["b_bn2"])

    pooled = _pool(xf, xb, batch)
    return _head(pooled, p["head1"], p["head2"])


# speedup vs baseline: 4.1515x; 1.0175x over previous
import functools

import jax
import jax.numpy as jnp
from jax import lax
from jax.experimental import pallas as pl
from jax.experimental.pallas import tpu as pltpu
from jax.experimental.pallas import tpu_sc as plsc

N_NODES = 50000
N_EDGES = 800000
HIDDEN = 64
N_GRAPHS = 512

F32 = jnp.float32


def _linear_body(x_ref, w_ref, b_ref, o_ref, *, relu):
    y = jnp.dot(x_ref[...], w_ref[...], preferred_element_type=F32) + b_ref[...]
    if relu:
        y = jnp.maximum(y, 0.0)
    o_ref[...] = y


def _linear(x, w, b, *, relu=False, block_rows):
    m, k = x.shape
    n = w.shape[1]
    assert m % block_rows == 0
    return pl.pallas_call(
        functools.partial(_linear_body, relu=relu),
        grid=(m // block_rows,),
        in_specs=[
            pl.BlockSpec((block_rows, k), lambda i: (i, 0)),
            pl.BlockSpec((k, n), lambda i: (0, 0)),
            pl.BlockSpec((1, n), lambda i: (0, 0)),
        ],
        out_specs=pl.BlockSpec((block_rows, n), lambda i: (i, 0)),
        out_shape=jax.ShapeDtypeStruct((m, n), F32),
    )(x, w, b.reshape(1, n))


def _mlp_stats_body(x_ref, a_ref, w1_ref, b1_ref, w2_ref, b2_ref,
                    z_ref, s_ref, ss_ref):
    h = x_ref[...] + a_ref[...]
    h1 = jnp.dot(h, w1_ref[...], preferred_element_type=F32) + b1_ref[...]
    h1 = jnp.maximum(h1, 0.0)
    z = jnp.dot(h1, w2_ref[...], preferred_element_type=F32) + b2_ref[...]
    z_ref[...] = z

    @pl.when(pl.program_id(0) == 0)
    def _zero():
        s_ref[...] = jnp.zeros_like(s_ref)
        ss_ref[...] = jnp.zeros_like(ss_ref)

    s_ref[...] += jnp.sum(z, axis=0, keepdims=True)
    ss_ref[...] += jnp.sum(z * z, axis=0, keepdims=True)


def _mlp_stats(x, aggr, p, *, block_rows=2000):
    m, d = x.shape
    assert m % block_rows == 0
    w1, b1 = p["l1"]["w"], p["l1"]["b"].reshape(1, d)
    w2, b2 = p["l2"]["w"], p["l2"]["b"].reshape(1, d)
    z, s, ss = pl.pallas_call(
        _mlp_stats_body,
        grid=(m // block_rows,),
        in_specs=[
            pl.BlockSpec((block_rows, d), lambda i: (i, 0)),
            pl.BlockSpec((block_rows, d), lambda i: (i, 0)),
            pl.BlockSpec((d, d), lambda i: (0, 0)),
            pl.BlockSpec((1, d), lambda i: (0, 0)),
            pl.BlockSpec((d, d), lambda i: (0, 0)),
            pl.BlockSpec((1, d), lambda i: (0, 0)),
        ],
        out_specs=[
            pl.BlockSpec((block_rows, d), lambda i: (i, 0)),
            pl.BlockSpec((1, d), lambda i: (0, 0)),
            pl.BlockSpec((1, d), lambda i: (0, 0)),
        ],
        out_shape=[
            jax.ShapeDtypeStruct((m, d), F32),
            jax.ShapeDtypeStruct((1, d), F32),
            jax.ShapeDtypeStruct((1, d), F32),
        ],
    )(x, aggr, w1, b1, w2, b2)
    return z, s, ss


def _bn_apply_body(z_ref, s_ref, ss_ref, g_ref, b_ref, o_ref, *, m):
    mean = s_ref[...] / m
    var = ss_ref[...] / m - mean * mean
    inv = jax.lax.rsqrt(var + 1e-5)
    o_ref[...] = jnp.maximum((z_ref[...] - mean) * inv * g_ref[...] + b_ref[...], 0.0)


def _bn_apply(z, s, ss, p, *, block_rows=2000):
    m, d = z.shape
    return pl.pallas_call(
        functools.partial(_bn_apply_body, m=float(m)),
        grid=(m // block_rows,),
        in_specs=[
            pl.BlockSpec((block_rows, d), lambda i: (i, 0)),
            pl.BlockSpec((1, d), lambda i: (0, 0)),
            pl.BlockSpec((1, d), lambda i: (0, 0)),
            pl.BlockSpec((1, d), lambda i: (0, 0)),
            pl.BlockSpec((1, d), lambda i: (0, 0)),
        ],
        out_specs=pl.BlockSpec((block_rows, d), lambda i: (i, 0)),
        out_shape=jax.ShapeDtypeStruct((m, d), F32),
    )(z, s, ss, p["g"].reshape(1, d), p["b"].reshape(1, d))


def _pool_body(xf_ref, xb_ref, batch_ref, o_ref):
    seg = batch_ref[...].reshape(1, -1).astype(jnp.int32)
    gids = jax.lax.broadcasted_iota(jnp.int32, (N_GRAPHS, seg.shape[1]), 0)
    onehot = (gids == seg).astype(F32)
    h = jnp.concatenate([xf_ref[...], xb_ref[...]], axis=1)

    @pl.when(pl.program_id(0) == 0)
    def _zero():
        o_ref[...] = jnp.zeros_like(o_ref)

    o_ref[...] += jnp.dot(onehot, h, preferred_element_type=F32)


def _pool(xf, xb, batch, *, block_rows=2000):
    m, d = xf.shape
    batch3 = batch.astype(jnp.int32).reshape(m // block_rows, 1, block_rows)
    return pl.pallas_call(
        _pool_body,
        grid=(m // block_rows,),
        in_specs=[
            pl.BlockSpec((block_rows, d), lambda i: (i, 0)),
            pl.BlockSpec((block_rows, d), lambda i: (i, 0)),
            pl.BlockSpec((1, 1, block_rows), lambda i: (i, 0, 0)),
        ],
        out_specs=pl.BlockSpec((N_GRAPHS, 2 * d), lambda i: (0, 0)),
        out_shape=jax.ShapeDtypeStruct((N_GRAPHS, 2 * d), F32),
    )(xf, xb, batch3)


def _head_body(x_ref, w1_ref, b1_ref, w2_ref, b2_ref, o_ref):
    h = jnp.dot(x_ref[...], w1_ref[...], preferred_element_type=F32) + b1_ref[...]
    h = jnp.maximum(h, 0.0)
    o_ref[...] = jnp.dot(h, w2_ref[...], preferred_element_type=F32) + b2_ref[...]


def _head(pooled, p1, p2):
    m, k = pooled.shape
    d = p1["w"].shape[1]
    out = pl.pallas_call(
        _head_body,
        in_specs=[
            pl.BlockSpec((m, k), lambda: (0, 0)),
            pl.BlockSpec((k, d), lambda: (0, 0)),
            pl.BlockSpec((1, d), lambda: (0, 0)),
            pl.BlockSpec((d, 1), lambda: (0, 0)),
            pl.BlockSpec((1, 1), lambda: (0, 0)),
        ],
        out_specs=pl.BlockSpec((m, 1), lambda: (0, 0)),
        out_shape=jax.ShapeDtypeStruct((m, 1), F32),
    )(pooled, p1["w"], p1["b"].reshape(1, d), p2["w"], p2["b"].reshape(1, 1))
    return out[:, 0]


_HALF = N_NODES // 2
_RPW = 1568
_SPM_ROWS = 16 * _RPW
_DUMP = _HALF
_EC = 80
_EPS = N_EDGES // 16
_NCH = _EPS // _EC
_ZR = _RPW // 32


_NBUF = 2


def _edge_aggr(t, ea, jj, ii):
    mesh = plsc.VectorSubcoreMesh(core_axis_name="c", subcore_axis_name="s")

    scratch = []
    for _ in range(_NBUF):
        scratch += [
            pltpu.VMEM((_EC,), jnp.int32),
            pltpu.VMEM((_EC,), jnp.int32),
            pltpu.VMEM((_EC,), jnp.int32),
            pltpu.VMEM((_EC, HIDDEN), F32),
            pltpu.VMEM((_EC, HIDDEN), F32),
        ]
    scratch += [
        pltpu.VMEM((_ZR, HIDDEN), F32),
        pltpu.VMEM_SHARED((_SPM_ROWS, HIDDEN), F32),
    ]
    scratch += [pltpu.SemaphoreType.DMA] * (5 * _NBUF)

    @functools.partial(
        pl.kernel,
        mesh=mesh,
        out_type=jax.ShapeDtypeStruct((N_NODES, HIDDEN), F32),
        compiler_params=pltpu.CompilerParams(use_tc_tiling_on_sc=False),
        scratch_types=scratch,
    )
    def k(t_hbm, ea_hbm, j_hbm, i_hbm, out_hbm, *refs):
        bufs = [refs[5 * b:5 * b + 5] for b in range(_NBUF)]
        zbuf = refs[5 * _NBUF]
        spm = refs[5 * _NBUF + 1]
        sems = refs[5 * _NBUF + 2:]
        sj = sems[0:_NBUF]
        si = sems[_NBUF:2 * _NBUF]
        sea = sems[2 * _NBUF:3 * _NBUF]
        sg = sems[3 * _NBUF:4 * _NBUF]
        ssc = sems[4 * _NBUF:5 * _NBUF]

        c = lax.axis_index("c")
        s = lax.axis_index("s")
        lo = c * _HALF
        ebase = s * _EPS
        n_iter = _NCH // _NBUF

        zv = jnp.zeros((16,), F32)

        @pl.loop(0, _ZR)
        def _zrow(r):
            for q in range(HIDDEN // 16):
                zbuf.at[r, pl.ds(q * 16, 16)][...] = zv

        @pl.loop(0, 32)
        def _zcopy(b):
            pltpu.sync_copy(zbuf, spm.at[pl.ds(s * _RPW + b * _ZR, _ZR)])

        plsc.subcore_barrier()

        def prefetch(b, kk):
            base = ebase + kk * _EC
            jb, ib, ilb, xjb, eab = bufs[b]
            pltpu.async_copy(j_hbm.at[pl.ds(base, _EC)], jb, sj[b])
            pltpu.async_copy(i_hbm.at[pl.ds(base, _EC)], ib, si[b])
            pltpu.async_copy(ea_hbm.at[pl.ds(base, _EC)], eab, sea[b])

        for b in range(_NBUF):
            prefetch(b, b)

        @pl.loop(0, n_iter)
        def _iter(k2):
            for b in range(_NBUF):
                jb, ib, ilb, xjb, eab = bufs[b]

                @pl.when(k2 > 0)
                def _drain():
                    pltpu.make_async_copy(xjb, spm.at[ilb], ssc[b]).wait()

                pltpu.make_async_copy(j_hbm.at[pl.ds(ebase, _EC)], jb,
                                      sj[b]).wait()
                pltpu.async_copy(t_hbm.at[jb], xjb, sg[b])

            for b in range(_NBUF):
                jb, ib, ilb, xjb, eab = bufs[b]
                kk = k2 * _NBUF + b
                base = ebase + kk * _EC

                pltpu.make_async_copy(i_hbm.at[pl.ds(ebase, _EC)], ib,
                                      si[b]).wait()

                @plsc.parallel_loop(0, _EC, step=16, unroll=2)
                def _locidx(q):
                    iv = ib.at[pl.ds(q, 16)][...]
                    ilv = iv - lo
                    valid = (ilv >= 0) & (ilv < _HALF)
                    dump = _DUMP + (iv & 63)
                    ilb.at[pl.ds(q, 16)][...] = jnp.where(valid, ilv, dump)

                pltpu.make_async_copy(ea_hbm.at[pl.ds(base, _EC)], eab,
                                      sea[b]).wait()
                pltpu.make_async_copy(t_hbm.at[jb], xjb, sg[b]).wait()

                @plsc.parallel_loop(0, _EC, unroll=4)
                def _msg(r):
                    for q in range(HIDDEN // 16):
                        sl = pl.ds(q * 16, 16)
                        v = xjb.at[r, sl][...] + eab.at[r, sl][...]
                        xjb.at[r, sl][...] = jnp.maximum(v, 0.0)

                pltpu.async_copy(xjb, spm.at[ilb], ssc[b], add=True)

                @pl.when(kk + _NBUF < n_iter * _NBUF)
                def _next():
                    prefetch(b, kk + _NBUF)

        for b in range(_NBUF):
            jb, ib, ilb, xjb, eab = bufs[b]
            pltpu.make_async_copy(xjb, spm.at[ilb], ssc[b]).wait()

        for kk in range(n_iter * _NBUF, _NCH):
            jb, ib, ilb, xjb, eab = bufs[0]
            base = ebase + kk * _EC
            pltpu.sync_copy(j_hbm.at[pl.ds(base, _EC)], jb)
            pltpu.sync_copy(i_hbm.at[pl.ds(base, _EC)], ib)
            pltpu.sync_copy(ea_hbm.at[pl.ds(base, _EC)], eab)
            pltpu.sync_copy(t_hbm.at[jb], xjb)

            @pl.loop(0, _EC, step=16)
            def _locidx_t(q):
                iv = ib.at[pl.ds(q, 16)][...]
                ilv = iv - lo
                valid = (ilv >= 0) & (ilv < _HALF)
                dump = _DUMP + (iv & 63)
                ilb.at[pl.ds(q, 16)][...] = jnp.where(valid, ilv, dump)

            @plsc.parallel_loop(0, _EC, unroll=4)
            def _msg_t(r):
                for q in range(HIDDEN // 16):
                    sl = pl.ds(q * 16, 16)
                    v = xjb.at[r, sl][...] + eab.at[r, sl][...]
                    xjb.at[r, sl][...] = jnp.maximum(v, 0.0)

            pltpu.sync_copy(xjb, spm.at[ilb], add=True)

        plsc.subcore_barrier()

        start = jnp.minimum(s * _RPW, _HALF - _RPW)
        pltpu.sync_copy(spm.at[pl.ds(start, _RPW)],
                        out_hbm.at[pl.ds(lo + start, _RPW)])

    return k(t, ea, jj, ii)


def _conv(t, ea, jj, ii, pmlp, pbn):
    aggr = _edge_aggr(t, ea, jj, ii)
    z, s, ss = _mlp_stats(t, aggr, pmlp)
    return _bn_apply(z, s, ss, pbn)


def kernel(x, edge_index, edge_attr, batch, params):
    p = params
    h = _linear(x, p["node_enc"]["w"], p["node_enc"]["b"], block_rows=2000)
    ea = _linear(edge_attr, p["edge_enc"]["w"], p["edge_enc"]["b"], block_rows=8000)
    src = edge_index[0]
    dst = edge_index[1]

    xf = _conv(h, ea, src, dst, p["f_conv1"], p["f_bn1"])
    xf = _conv(xf, ea, src, dst, p["f_conv2"], p["f_bn2"])
    xb = _conv(h, ea, dst, src, p["b_conv1"], p["b_bn1"])
    xb = _conv(xb, ea, dst, src, p["b_conv2"], p["b_bn2"])

    pooled = _pool(xf, xb, batch)
    return _head(pooled, p["head1"], p["head2"])


# speedup vs baseline: 5.6390x; 1.3583x over previous
import dataclasses
import functools

import jax
import jax.numpy as jnp
from jax import lax
from jax.experimental import pallas as pl
from jax.experimental.pallas import tpu as pltpu
from jax.experimental.pallas import tpu_sc as plsc

N_NODES = 50000
N_EDGES = 800000
HIDDEN = 64
N_GRAPHS = 512

F32 = jnp.float32


def _linear_body(x_ref, w_ref, b_ref, o_ref, *, relu):
    y = jnp.dot(x_ref[...], w_ref[...], preferred_element_type=F32) + b_ref[...]
    if relu:
        y = jnp.maximum(y, 0.0)
    o_ref[...] = y


def _linear(x, w, b, *, relu=False, block_rows):
    m, k = x.shape
    n = w.shape[1]
    assert m % block_rows == 0
    return pl.pallas_call(
        functools.partial(_linear_body, relu=relu),
        grid=(m // block_rows,),
        in_specs=[
            pl.BlockSpec((block_rows, k), lambda i: (i, 0)),
            pl.BlockSpec((k, n), lambda i: (0, 0)),
            pl.BlockSpec((1, n), lambda i: (0, 0)),
        ],
        out_specs=pl.BlockSpec((block_rows, n), lambda i: (i, 0)),
        out_shape=jax.ShapeDtypeStruct((m, n), F32),
    )(x, w, b.reshape(1, n))


def _mlp_stats_body(x_ref, a_ref, w1_ref, b1_ref, w2_ref, b2_ref,
                    z_ref, s_ref, ss_ref):
    h = x_ref[...] + a_ref[...]
    h1 = jnp.dot(h, w1_ref[...], preferred_element_type=F32) + b1_ref[...]
    h1 = jnp.maximum(h1, 0.0)
    z = jnp.dot(h1, w2_ref[...], preferred_element_type=F32) + b2_ref[...]
    z_ref[...] = z

    @pl.when(pl.program_id(0) == 0)
    def _zero():
        s_ref[...] = jnp.zeros_like(s_ref)
        ss_ref[...] = jnp.zeros_like(ss_ref)

    s_ref[...] += jnp.sum(z, axis=0, keepdims=True)
    ss_ref[...] += jnp.sum(z * z, axis=0, keepdims=True)


def _mlp_stats(x, aggr, p, *, block_rows=2000):
    m, d = x.shape
    assert m % block_rows == 0
    w1, b1 = p["l1"]["w"], p["l1"]["b"].reshape(1, d)
    w2, b2 = p["l2"]["w"], p["l2"]["b"].reshape(1, d)
    z, s, ss = pl.pallas_call(
        _mlp_stats_body,
        grid=(m // block_rows,),
        in_specs=[
            pl.BlockSpec((block_rows, d), lambda i: (i, 0)),
            pl.BlockSpec((block_rows, d), lambda i: (i, 0)),
            pl.BlockSpec((d, d), lambda i: (0, 0)),
            pl.BlockSpec((1, d), lambda i: (0, 0)),
            pl.BlockSpec((d, d), lambda i: (0, 0)),
            pl.BlockSpec((1, d), lambda i: (0, 0)),
        ],
        out_specs=[
            pl.BlockSpec((block_rows, d), lambda i: (i, 0)),
            pl.BlockSpec((1, d), lambda i: (0, 0)),
            pl.BlockSpec((1, d), lambda i: (0, 0)),
        ],
        out_shape=[
            jax.ShapeDtypeStruct((m, d), F32),
            jax.ShapeDtypeStruct((1, d), F32),
            jax.ShapeDtypeStruct((1, d), F32),
        ],
    )(x, aggr, w1, b1, w2, b2)
    return z, s, ss


def _bn_apply_body(z_ref, s_ref, ss_ref, g_ref, b_ref, o_ref, *, m):
    mean = s_ref[...] / m
    var = ss_ref[...] / m - mean * mean
    inv = jax.lax.rsqrt(var + 1e-5)
    o_ref[...] = jnp.maximum((z_ref[...] - mean) * inv * g_ref[...] + b_ref[...], 0.0)


def _bn_apply(z, s, ss, p, *, block_rows=2000):
    m, d = z.shape
    return pl.pallas_call(
        functools.partial(_bn_apply_body, m=float(m)),
        grid=(m // block_rows,),
        in_specs=[
            pl.BlockSpec((block_rows, d), lambda i: (i, 0)),
            pl.BlockSpec((1, d), lambda i: (0, 0)),
            pl.BlockSpec((1, d), lambda i: (0, 0)),
            pl.BlockSpec((1, d), lambda i: (0, 0)),
            pl.BlockSpec((1, d), lambda i: (0, 0)),
        ],
        out_specs=pl.BlockSpec((block_rows, d), lambda i: (i, 0)),
        out_shape=jax.ShapeDtypeStruct((m, d), F32),
    )(z, s, ss, p["g"].reshape(1, d), p["b"].reshape(1, d))


def _pool_body(xf_ref, xb_ref, batch_ref, o_ref):
    seg = batch_ref[...].reshape(1, -1).astype(jnp.int32)
    gids = jax.lax.broadcasted_iota(jnp.int32, (N_GRAPHS, seg.shape[1]), 0)
    onehot = (gids == seg).astype(F32)
    h = jnp.concatenate([xf_ref[...], xb_ref[...]], axis=1)

    @pl.when(pl.program_id(0) == 0)
    def _zero():
        o_ref[...] = jnp.zeros_like(o_ref)

    o_ref[...] += jnp.dot(onehot, h, preferred_element_type=F32)


def _pool(xf, xb, batch, *, block_rows=2000):
    m, d = xf.shape
    batch3 = batch.astype(jnp.int32).reshape(m // block_rows, 1, block_rows)
    return pl.pallas_call(
        _pool_body,
        grid=(m // block_rows,),
        in_specs=[
            pl.BlockSpec((block_rows, d), lambda i: (i, 0)),
            pl.BlockSpec((block_rows, d), lambda i: (i, 0)),
            pl.BlockSpec((1, 1, block_rows), lambda i: (i, 0, 0)),
        ],
        out_specs=pl.BlockSpec((N_GRAPHS, 2 * d), lambda i: (0, 0)),
        out_shape=jax.ShapeDtypeStruct((N_GRAPHS, 2 * d), F32),
    )(xf, xb, batch3)


def _head_body(x_ref, w1_ref, b1_ref, w2_ref, b2_ref, o_ref):
    h = jnp.dot(x_ref[...], w1_ref[...], preferred_element_type=F32) + b1_ref[...]
    h = jnp.maximum(h, 0.0)
    o_ref[...] = jnp.dot(h, w2_ref[...], preferred_element_type=F32) + b2_ref[...]


def _head(pooled, p1, p2):
    m, k = pooled.shape
    d = p1["w"].shape[1]
    out = pl.pallas_call(
        _head_body,
        in_specs=[
            pl.BlockSpec((m, k), lambda: (0, 0)),
            pl.BlockSpec((k, d), lambda: (0, 0)),
            pl.BlockSpec((1, d), lambda: (0, 0)),
            pl.BlockSpec((d, 1), lambda: (0, 0)),
            pl.BlockSpec((1, 1), lambda: (0, 0)),
        ],
        out_specs=pl.BlockSpec((m, 1), lambda: (0, 0)),
        out_shape=jax.ShapeDtypeStruct((m, 1), F32),
    )(pooled, p1["w"], p1["b"].reshape(1, d), p2["w"], p2["b"].reshape(1, 1))
    return out[:, 0]


_HALF = N_NODES // 2
_RPW = 1568
_SPM_ROWS = 16 * _RPW
_DUMP = _HALF
_EC = 80
_EPS = N_EDGES // 16
_NCH = _EPS // _EC
_ZR = _RPW // 32


_NBUF = 2

_FB = 960
_FFIN = 1152
_CAPW = 52 * _FB + _FFIN
_CBUF = 1280
_RCH = 10000


def _route(jarr, iarr):
    mesh = plsc.VectorSubcoreMesh(core_axis_name="c", subcore_axis_name="s")

    out_type = [
        jax.ShapeDtypeStruct((2, 16, _CAPW), jnp.int32),
        jax.ShapeDtypeStruct((2, 16, _CAPW), jnp.int32),
        jax.ShapeDtypeStruct((2, 16, _CAPW), jnp.int32),
        jax.ShapeDtypeStruct((2, 16, 16), jnp.int32),
    ]
    scratch = [
        pltpu.VMEM((_RCH,), jnp.int32),
        pltpu.VMEM((_RCH,), jnp.int32),
        pltpu.VMEM((_CBUF,), jnp.int32),
        pltpu.VMEM((_CBUF,), jnp.int32),
        pltpu.VMEM((_CBUF,), jnp.int32),
        pltpu.VMEM((16,), jnp.int32),
    ]

    cp = pltpu.CompilerParams(use_tc_tiling_on_sc=False)
    if "needs_layout_passes" in pltpu.CompilerParams.__dataclass_fields__:
        cp = dataclasses.replace(cp, needs_layout_passes=False)

    @functools.partial(
        pl.kernel,
        mesh=mesh,
        out_type=out_type,
        compiler_params=cp,
        scratch_types=scratch,
    )
    def k(j_hbm, i_hbm, jr, er, ir, counts, jc, ic, bj, be, bi, cntb):
        c = lax.axis_index("c")
        s = lax.axis_index("s")
        lo = c * _HALF
        lanes = lax.iota(jnp.int32, 16)

        def group(g, carry, chunk_base):
            cur, nblk = carry
            off = g * 16
            iv = ic.at[pl.ds(off, 16)][...]
            jv = jc.at[pl.ds(off, 16)][...]
            ev = chunk_base + off + lanes
            ilv = iv - lo
            m = (ilv >= 0) & (ilv < _HALF)
            plsc.store_compressed(bj.at[pl.ds(cur, 16)], jv, mask=m)
            plsc.store_compressed(be.at[pl.ds(cur, 16)], ev, mask=m)
            plsc.store_compressed(bi.at[pl.ds(cur, 16)], ilv, mask=m)
            cnt = jnp.max(plsc.all_reduce_population_count(m))
            cur = cur + cnt
            flush = cur >= _FB

            @pl.when(flush)
            def _flush():
                pltpu.sync_copy(bj.at[pl.ds(0, _FB)],
                                jr.at[c, s, pl.ds(nblk * _FB, _FB)])
                pltpu.sync_copy(be.at[pl.ds(0, _FB)],
                                er.at[c, s, pl.ds(nblk * _FB, _FB)])
                pltpu.sync_copy(bi.at[pl.ds(0, _FB)],
                                ir.at[c, s, pl.ds(nblk * _FB, _FB)])
                bj.at[pl.ds(0, 16)][...] = bj.at[pl.ds(_FB, 16)][...]
                be.at[pl.ds(0, 16)][...] = be.at[pl.ds(_FB, 16)][...]
                bi.at[pl.ds(0, 16)][...] = bi.at[pl.ds(_FB, 16)][...]

            cur = jnp.where(flush, cur - _FB, cur)
            nblk = nblk + flush.astype(jnp.int32)
            return cur, nblk

        carry = (jnp.int32(0), jnp.int32(0))
        for ch in range(_EPS // _RCH):
            base = s * _EPS + ch * _RCH
            pltpu.sync_copy(j_hbm.at[pl.ds(base, _RCH)], jc)
            pltpu.sync_copy(i_hbm.at[pl.ds(base, _RCH)], ic)
            carry = lax.fori_loop(
                0, _RCH // 16,
                functools.partial(group, chunk_base=base), carry)
        cur, nblk = carry

        zv16 = jnp.zeros((16,), jnp.int32)
        dumpv = _DUMP + lanes

        @plsc.parallel_loop(0, 160, step=16)
        def _pad(kq):
            bj.at[pl.ds(cur + kq, 16)][...] = zv16
            be.at[pl.ds(cur + kq, 16)][...] = zv16
            bi.at[pl.ds(cur + kq, 16)][...] = dumpv

        cur_pad = jnp.maximum(((cur + _EC - 1) // _EC) * _EC, 2 * _EC)
        pltpu.sync_copy(bj.at[pl.ds(0, _FFIN)],
                        jr.at[c, s, pl.ds(nblk * _FB, _FFIN)])
        pltpu.sync_copy(be.at[pl.ds(0, _FFIN)],
                        er.at[c, s, pl.ds(nblk * _FB, _FFIN)])
        pltpu.sync_copy(bi.at[pl.ds(0, _FFIN)],
                        ir.at[c, s, pl.ds(nblk * _FB, _FFIN)])

        total = nblk * _FB + cur_pad
        cntb[...] = jnp.where(lanes == 0, total, 0)
        pltpu.sync_copy(cntb, counts.at[c, s])

    return k(jarr, iarr)


def _edge_aggr(t, ea, jr, er, ir, counts):
    mesh = plsc.VectorSubcoreMesh(core_axis_name="c", subcore_axis_name="s")

    scratch = []
    for _ in range(_NBUF):
        scratch += [
            pltpu.VMEM((_EC,), jnp.int32),
            pltpu.VMEM((_EC,), jnp.int32),
            pltpu.VMEM((_EC,), jnp.int32),
            pltpu.VMEM((_EC, HIDDEN), F32),
            pltpu.VMEM((_EC, HIDDEN), F32),
        ]
    scratch += [
        pltpu.VMEM((_ZR, HIDDEN), F32),
        pltpu.VMEM_SHARED((_SPM_ROWS, HIDDEN), F32),
        pltpu.VMEM((16,), jnp.int32),
    ]
    scratch += [pltpu.SemaphoreType.DMA] * (6 * _NBUF)

    cp = pltpu.CompilerParams(use_tc_tiling_on_sc=False)
    if "needs_layout_passes" in pltpu.CompilerParams.__dataclass_fields__:
        cp = dataclasses.replace(cp, needs_layout_passes=False)

    @functools.partial(
        pl.kernel,
        mesh=mesh,
        out_type=jax.ShapeDtypeStruct((N_NODES, HIDDEN), F32),
        compiler_params=cp,
        scratch_types=scratch,
    )
    def k(t_hbm, ea_hbm, jr_hbm, er_hbm, ir_hbm, cnt_hbm, out_hbm, *refs):
        bufs = [refs[5 * b:5 * b + 5] for b in range(_NBUF)]
        zbuf = refs[5 * _NBUF]
        spm = refs[5 * _NBUF + 1]
        cntb = refs[5 * _NBUF + 2]
        sems = refs[5 * _NBUF + 3:]
        sjr = sems[0:_NBUF]
        ser = sems[_NBUF:2 * _NBUF]
        sir = sems[2 * _NBUF:3 * _NBUF]
        sg = sems[3 * _NBUF:4 * _NBUF]
        sge = sems[4 * _NBUF:5 * _NBUF]
        ssc = sems[5 * _NBUF:6 * _NBUF]

        c = lax.axis_index("c")
        s = lax.axis_index("s")
        lo = c * _HALF

        pltpu.sync_copy(cnt_hbm.at[c, s], cntb)
        n80 = jnp.max(cntb[...]) // _EC
        n2 = n80 // _NBUF

        zv = jnp.zeros((16,), F32)

        @pl.loop(0, _ZR)
        def _zrow(r):
            for q in range(HIDDEN // 16):
                zbuf.at[r, pl.ds(q * 16, 16)][...] = zv

        @pl.loop(0, 32)
        def _zcopy(b):
            pltpu.sync_copy(zbuf, spm.at[pl.ds(s * _RPW + b * _ZR, _ZR)])

        plsc.subcore_barrier()

        def prefetch(b, kk):
            jb, eb, ilb, xjb, eab = bufs[b]
            pltpu.async_copy(jr_hbm.at[c, s, pl.ds(kk * _EC, _EC)], jb, sjr[b])
            pltpu.async_copy(er_hbm.at[c, s, pl.ds(kk * _EC, _EC)], eb, ser[b])
            pltpu.async_copy(ir_hbm.at[c, s, pl.ds(kk * _EC, _EC)], ilb, sir[b])

        for b in range(_NBUF):
            prefetch(b, b)

        @pl.loop(0, n2)
        def _iter(k2):
            for b in range(_NBUF):
                jb, eb, ilb, xjb, eab = bufs[b]

                @pl.when(k2 > 0)
                def _drain():
                    pltpu.make_async_copy(xjb, spm.at[ilb], ssc[b]).wait()

                pltpu.make_async_copy(jr_hbm.at[c, s, pl.ds(0, _EC)], jb,
                                      sjr[b]).wait()
                pltpu.async_copy(t_hbm.at[jb], xjb, sg[b])
                pltpu.make_async_copy(er_hbm.at[c, s, pl.ds(0, _EC)], eb,
                                      ser[b]).wait()
                pltpu.async_copy(ea_hbm.at[eb], eab, sge[b])

            for b in range(_NBUF):
                jb, eb, ilb, xjb, eab = bufs[b]
                kk = k2 * _NBUF + b

                pltpu.make_async_copy(ir_hbm.at[c, s, pl.ds(0, _EC)], ilb,
                                      sir[b]).wait()
                pltpu.make_async_copy(t_hbm.at[jb], xjb, sg[b]).wait()
                pltpu.make_async_copy(ea_hbm.at[eb], eab, sge[b]).wait()

                @plsc.parallel_loop(0, _EC, unroll=4)
                def _msg(r):
                    for q in range(HIDDEN // 16):
                        sl = pl.ds(q * 16, 16)
                        v = xjb.at[r, sl][...] + eab.at[r, sl][...]
                        xjb.at[r, sl][...] = jnp.maximum(v, 0.0)

                pltpu.async_copy(xjb, spm.at[ilb], ssc[b], add=True)

                @pl.when(kk + _NBUF < n2 * _NBUF)
                def _next():
                    prefetch(b, kk + _NBUF)

        for b in range(_NBUF):
            jb, eb, ilb, xjb, eab = bufs[b]
            pltpu.make_async_copy(xjb, spm.at[ilb], ssc[b]).wait()

        @pl.when(n80 > n2 * _NBUF)
        def _tail():
            jb, eb, ilb, xjb, eab = bufs[0]
            kk = n80 - 1
            pltpu.sync_copy(jr_hbm.at[c, s, pl.ds(kk * _EC, _EC)], jb)
            pltpu.sync_copy(er_hbm.at[c, s, pl.ds(kk * _EC, _EC)], eb)
            pltpu.sync_copy(ir_hbm.at[c, s, pl.ds(kk * _EC, _EC)], ilb)
            pltpu.sync_copy(t_hbm.at[jb], xjb)
            pltpu.sync_copy(ea_hbm.at[eb], eab)

            @plsc.parallel_loop(0, _EC, unroll=4)
            def _msg_t(r):
                for q in range(HIDDEN // 16):
                    sl = pl.ds(q * 16, 16)
                    v = xjb.at[r, sl][...] + eab.at[r, sl][...]
                    xjb.at[r, sl][...] = jnp.maximum(v, 0.0)

            pltpu.sync_copy(xjb, spm.at[ilb], add=True)

        plsc.subcore_barrier()

        start = jnp.minimum(s * _RPW, _HALF - _RPW)
        pltpu.sync_copy(spm.at[pl.ds(start, _RPW)],
                        out_hbm.at[pl.ds(lo + start, _RPW)])

    return k(t, ea, jr, er, ir, counts)


def _conv(t, ea, route, pmlp, pbn):
    aggr = _edge_aggr(t, ea, *route)
    z, s, ss = _mlp_stats(t, aggr, pmlp)
    return _bn_apply(z, s, ss, pbn)


def kernel(x, edge_index, edge_attr, batch, params):
    p = params
    h = _linear(x, p["node_enc"]["w"], p["node_enc"]["b"], block_rows=2000)
    ea = _linear(edge_attr, p["edge_enc"]["w"], p["edge_enc"]["b"], block_rows=8000)
    src = edge_index[0]
    dst = edge_index[1]

    route_f = _route(src, dst)
    route_b = _route(dst, src)

    xf = _conv(h, ea, route_f, p["f_conv1"], p["f_bn1"])
    xf = _conv(xf, ea, route_f, p["f_conv2"], p["f_bn2"])
    xb = _conv(h, ea, route_b, p["b_conv1"], p["b_bn1"])
    xb = _conv(xb, ea, route_b, p["b_conv2"], p["b_bn2"])

    pooled = _pool(xf, xb, batch)
    return _head(pooled, p["head1"], p["head2"])


# speedup vs baseline: 5.6675x; 1.0051x over previous
import dataclasses
import functools

import jax
import jax.numpy as jnp
from jax import lax
from jax.experimental import pallas as pl
from jax.experimental.pallas import tpu as pltpu
from jax.experimental.pallas import tpu_sc as plsc

N_NODES = 50000
N_EDGES = 800000
HIDDEN = 64
N_GRAPHS = 512

F32 = jnp.float32


def _linear_body(x_ref, w_ref, b_ref, o_ref, *, relu):
    y = jnp.dot(x_ref[...], w_ref[...], preferred_element_type=F32) + b_ref[...]
    if relu:
        y = jnp.maximum(y, 0.0)
    o_ref[...] = y


def _linear(x, w, b, *, relu=False, block_rows):
    m, k = x.shape
    n = w.shape[1]
    assert m % block_rows == 0
    return pl.pallas_call(
        functools.partial(_linear_body, relu=relu),
        grid=(m // block_rows,),
        in_specs=[
            pl.BlockSpec((block_rows, k), lambda i: (i, 0)),
            pl.BlockSpec((k, n), lambda i: (0, 0)),
            pl.BlockSpec((1, n), lambda i: (0, 0)),
        ],
        out_specs=pl.BlockSpec((block_rows, n), lambda i: (i, 0)),
        out_shape=jax.ShapeDtypeStruct((m, n), F32),
    )(x, w, b.reshape(1, n))


def _mlp_stats_body(x_ref, a_ref, w1_ref, b1_ref, w2_ref, b2_ref,
                    z_ref, s_ref, ss_ref):
    h = x_ref[...] + a_ref[...]
    h1 = jnp.dot(h, w1_ref[...], preferred_element_type=F32) + b1_ref[...]
    h1 = jnp.maximum(h1, 0.0)
    z = jnp.dot(h1, w2_ref[...], preferred_element_type=F32) + b2_ref[...]
    z_ref[...] = z

    @pl.when(pl.program_id(0) == 0)
    def _zero():
        s_ref[...] = jnp.zeros_like(s_ref)
        ss_ref[...] = jnp.zeros_like(ss_ref)

    s_ref[...] += jnp.sum(z, axis=0, keepdims=True)
    ss_ref[...] += jnp.sum(z * z, axis=0, keepdims=True)


def _mlp_stats(x, aggr, p, *, block_rows=2000):
    m, d = x.shape
    assert m % block_rows == 0
    w1, b1 = p["l1"]["w"], p["l1"]["b"].reshape(1, d)
    w2, b2 = p["l2"]["w"], p["l2"]["b"].reshape(1, d)
    z, s, ss = pl.pallas_call(
        _mlp_stats_body,
        grid=(m // block_rows,),
        in_specs=[
            pl.BlockSpec((block_rows, d), lambda i: (i, 0)),
            pl.BlockSpec((block_rows, d), lambda i: (i, 0)),
            pl.BlockSpec((d, d), lambda i: (0, 0)),
            pl.BlockSpec((1, d), lambda i: (0, 0)),
            pl.BlockSpec((d, d), lambda i: (0, 0)),
            pl.BlockSpec((1, d), lambda i: (0, 0)),
        ],
        out_specs=[
            pl.BlockSpec((block_rows, d), lambda i: (i, 0)),
            pl.BlockSpec((1, d), lambda i: (0, 0)),
            pl.BlockSpec((1, d), lambda i: (0, 0)),
        ],
        out_shape=[
            jax.ShapeDtypeStruct((m, d), F32),
            jax.ShapeDtypeStruct((1, d), F32),
            jax.ShapeDtypeStruct((1, d), F32),
        ],
    )(x, aggr, w1, b1, w2, b2)
    return z, s, ss


def _bn_apply_body(z_ref, s_ref, ss_ref, g_ref, b_ref, o_ref, *, m):
    mean = s_ref[...] / m
    var = ss_ref[...] / m - mean * mean
    inv = jax.lax.rsqrt(var + 1e-5)
    o_ref[...] = jnp.maximum((z_ref[...] - mean) * inv * g_ref[...] + b_ref[...], 0.0)


def _bn_apply(z, s, ss, p, *, block_rows=2000):
    m, d = z.shape
    return pl.pallas_call(
        functools.partial(_bn_apply_body, m=float(m)),
        grid=(m // block_rows,),
        in_specs=[
            pl.BlockSpec((block_rows, d), lambda i: (i, 0)),
            pl.BlockSpec((1, d), lambda i: (0, 0)),
            pl.BlockSpec((1, d), lambda i: (0, 0)),
            pl.BlockSpec((1, d), lambda i: (0, 0)),
            pl.BlockSpec((1, d), lambda i: (0, 0)),
        ],
        out_specs=pl.BlockSpec((block_rows, d), lambda i: (i, 0)),
        out_shape=jax.ShapeDtypeStruct((m, d), F32),
    )(z, s, ss, p["g"].reshape(1, d), p["b"].reshape(1, d))


def _pool_body(xf_ref, xb_ref, batch_ref, o_ref):
    seg = batch_ref[...].reshape(1, -1).astype(jnp.int32)
    gids = jax.lax.broadcasted_iota(jnp.int32, (N_GRAPHS, seg.shape[1]), 0)
    onehot = (gids == seg).astype(F32)
    h = jnp.concatenate([xf_ref[...], xb_ref[...]], axis=1)

    @pl.when(pl.program_id(0) == 0)
    def _zero():
        o_ref[...] = jnp.zeros_like(o_ref)

    o_ref[...] += jnp.dot(onehot, h, preferred_element_type=F32)


def _pool(xf, xb, batch, *, block_rows=2000):
    m, d = xf.shape
    batch3 = batch.astype(jnp.int32).reshape(m // block_rows, 1, block_rows)
    return pl.pallas_call(
        _pool_body,
        grid=(m // block_rows,),
        in_specs=[
            pl.BlockSpec((block_rows, d), lambda i: (i, 0)),
            pl.BlockSpec((block_rows, d), lambda i: (i, 0)),
            pl.BlockSpec((1, 1, block_rows), lambda i: (i, 0, 0)),
        ],
        out_specs=pl.BlockSpec((N_GRAPHS, 2 * d), lambda i: (0, 0)),
        out_shape=jax.ShapeDtypeStruct((N_GRAPHS, 2 * d), F32),
    )(xf, xb, batch3)


def _head_body(x_ref, w1_ref, b1_ref, w2_ref, b2_ref, o_ref):
    h = jnp.dot(x_ref[...], w1_ref[...], preferred_element_type=F32) + b1_ref[...]
    h = jnp.maximum(h, 0.0)
    o_ref[...] = jnp.dot(h, w2_ref[...], preferred_element_type=F32) + b2_ref[...]


def _head(pooled, p1, p2):
    m, k = pooled.shape
    d = p1["w"].shape[1]
    out = pl.pallas_call(
        _head_body,
        in_specs=[
            pl.BlockSpec((m, k), lambda: (0, 0)),
            pl.BlockSpec((k, d), lambda: (0, 0)),
            pl.BlockSpec((1, d), lambda: (0, 0)),
            pl.BlockSpec((d, 1), lambda: (0, 0)),
            pl.BlockSpec((1, 1), lambda: (0, 0)),
        ],
        out_specs=pl.BlockSpec((m, 1), lambda: (0, 0)),
        out_shape=jax.ShapeDtypeStruct((m, 1), F32),
    )(pooled, p1["w"], p1["b"].reshape(1, d), p2["w"], p2["b"].reshape(1, 1))
    return out[:, 0]


_HALF = N_NODES // 2
_RPW = 1568
_SPM_ROWS = 16 * _RPW
_DUMP = _HALF
_EC = 96
_EPS = N_EDGES // 16
_NCH = _EPS // _EC
_ZR = _RPW // 32


_NBUF = 2

_FB = 960
_FFIN = 1216
_CAPW = 52 * _FB + _FFIN
_CBUF = 1280
_RCH = 10000


def _route(jarr, iarr):
    mesh = plsc.VectorSubcoreMesh(core_axis_name="c", subcore_axis_name="s")

    out_type = [
        jax.ShapeDtypeStruct((2, 16, _CAPW), jnp.int32),
        jax.ShapeDtypeStruct((2, 16, _CAPW), jnp.int32),
        jax.ShapeDtypeStruct((2, 16, _CAPW), jnp.int32),
        jax.ShapeDtypeStruct((2, 16, 16), jnp.int32),
    ]
    scratch = [
        pltpu.VMEM((_RCH,), jnp.int32),
        pltpu.VMEM((_RCH,), jnp.int32),
        pltpu.VMEM((_CBUF,), jnp.int32),
        pltpu.VMEM((_CBUF,), jnp.int32),
        pltpu.VMEM((_CBUF,), jnp.int32),
        pltpu.VMEM((16,), jnp.int32),
    ]

    cp = pltpu.CompilerParams(use_tc_tiling_on_sc=False)
    if "needs_layout_passes" in pltpu.CompilerParams.__dataclass_fields__:
        cp = dataclasses.replace(cp, needs_layout_passes=False)

    @functools.partial(
        pl.kernel,
        mesh=mesh,
        out_type=out_type,
        compiler_params=cp,
        scratch_types=scratch,
    )
    def k(j_hbm, i_hbm, jr, er, ir, counts, jc, ic, bj, be, bi, cntb):
        c = lax.axis_index("c")
        s = lax.axis_index("s")
        lo = c * _HALF
        lanes = lax.iota(jnp.int32, 16)

        def group(g, carry, chunk_base):
            cur, nblk = carry
            off = g * 16
            iv = ic.at[pl.ds(off, 16)][...]
            jv = jc.at[pl.ds(off, 16)][...]
            ev = chunk_base + off + lanes
            ilv = iv - lo
            m = (ilv >= 0) & (ilv < _HALF)
            plsc.store_compressed(bj.at[pl.ds(cur, 16)], jv, mask=m)
            plsc.store_compressed(be.at[pl.ds(cur, 16)], ev, mask=m)
            plsc.store_compressed(bi.at[pl.ds(cur, 16)], ilv, mask=m)
            cnt = jnp.max(plsc.all_reduce_population_count(m))
            cur = cur + cnt
            flush = cur >= _FB

            @pl.when(flush)
            def _flush():
                pltpu.sync_copy(bj.at[pl.ds(0, _FB)],
                                jr.at[c, s, pl.ds(nblk * _FB, _FB)])
                pltpu.sync_copy(be.at[pl.ds(0, _FB)],
                                er.at[c, s, pl.ds(nblk * _FB, _FB)])
                pltpu.sync_copy(bi.at[pl.ds(0, _FB)],
                                ir.at[c, s, pl.ds(nblk * _FB, _FB)])
                bj.at[pl.ds(0, 16)][...] = bj.at[pl.ds(_FB, 16)][...]
                be.at[pl.ds(0, 16)][...] = be.at[pl.ds(_FB, 16)][...]
                bi.at[pl.ds(0, 16)][...] = bi.at[pl.ds(_FB, 16)][...]

            cur = jnp.where(flush, cur - _FB, cur)
            nblk = nblk + flush.astype(jnp.int32)
            return cur, nblk

        carry = (jnp.int32(0), jnp.int32(0))
        for ch in range(_EPS // _RCH):
            base = s * _EPS + ch * _RCH
            pltpu.sync_copy(j_hbm.at[pl.ds(base, _RCH)], jc)
            pltpu.sync_copy(i_hbm.at[pl.ds(base, _RCH)], ic)
            carry = lax.fori_loop(
                0, _RCH // 16,
                functools.partial(group, chunk_base=base), carry)
        cur, nblk = carry

        zv16 = jnp.zeros((16,), jnp.int32)
        dumpv = _DUMP + lanes

        @plsc.parallel_loop(0, 2 * _EC, step=16)
        def _pad(kq):
            bj.at[pl.ds(cur + kq, 16)][...] = zv16
            be.at[pl.ds(cur + kq, 16)][...] = zv16
            bi.at[pl.ds(cur + kq, 16)][...] = dumpv

        cur_pad = jnp.maximum(((cur + _EC - 1) // _EC) * _EC, 2 * _EC)
        pltpu.sync_copy(bj.at[pl.ds(0, _FFIN)],
                        jr.at[c, s, pl.ds(nblk * _FB, _FFIN)])
        pltpu.sync_copy(be.at[pl.ds(0, _FFIN)],
                        er.at[c, s, pl.ds(nblk * _FB, _FFIN)])
        pltpu.sync_copy(bi.at[pl.ds(0, _FFIN)],
                        ir.at[c, s, pl.ds(nblk * _FB, _FFIN)])

        total = nblk * _FB + cur_pad
        cntb[...] = jnp.where(lanes == 0, total, 0)
        pltpu.sync_copy(cntb, counts.at[c, s])

    return k(jarr, iarr)


def _edge_aggr(t, ea, jr, er, ir, counts):
    mesh = plsc.VectorSubcoreMesh(core_axis_name="c", subcore_axis_name="s")

    scratch = []
    for _ in range(_NBUF):
        scratch += [
            pltpu.VMEM((_EC,), jnp.int32),
            pltpu.VMEM((_EC,), jnp.int32),
            pltpu.VMEM((_EC,), jnp.int32),
            pltpu.VMEM((_EC, HIDDEN), F32),
            pltpu.VMEM((_EC, HIDDEN), F32),
        ]
    scratch += [
        pltpu.VMEM((_ZR, HIDDEN), F32),
        pltpu.VMEM_SHARED((_SPM_ROWS, HIDDEN), F32),
        pltpu.VMEM((16,), jnp.int32),
    ]
    scratch += [pltpu.SemaphoreType.DMA] * (6 * _NBUF)

    cp = pltpu.CompilerParams(use_tc_tiling_on_sc=False)
    if "needs_layout_passes" in pltpu.CompilerParams.__dataclass_fields__:
        cp = dataclasses.replace(cp, needs_layout_passes=False)

    @functools.partial(
        pl.kernel,
        mesh=mesh,
        out_type=jax.ShapeDtypeStruct((N_NODES, HIDDEN), F32),
        compiler_params=cp,
        scratch_types=scratch,
    )
    def k(t_hbm, ea_hbm, jr_hbm, er_hbm, ir_hbm, cnt_hbm, out_hbm, *refs):
        bufs = [refs[5 * b:5 * b + 5] for b in range(_NBUF)]
        zbuf = refs[5 * _NBUF]
        spm = refs[5 * _NBUF + 1]
        cntb = refs[5 * _NBUF + 2]
        sems = refs[5 * _NBUF + 3:]
        sjr = sems[0:_NBUF]
        ser = sems[_NBUF:2 * _NBUF]
        sir = sems[2 * _NBUF:3 * _NBUF]
        sg = sems[3 * _NBUF:4 * _NBUF]
        sge = sems[4 * _NBUF:5 * _NBUF]
        ssc = sems[5 * _NBUF:6 * _NBUF]

        c = lax.axis_index("c")
        s = lax.axis_index("s")
        lo = c * _HALF

        pltpu.sync_copy(cnt_hbm.at[c, s], cntb)
        n80 = jnp.max(cntb[...]) // _EC
        n2 = n80 // _NBUF

        zv = jnp.zeros((16,), F32)

        @pl.loop(0, _ZR)
        def _zrow(r):
            for q in range(HIDDEN // 16):
                zbuf.at[r, pl.ds(q * 16, 16)][...] = zv

        @pl.loop(0, 32)
        def _zcopy(b):
            pltpu.sync_copy(zbuf, spm.at[pl.ds(s * _RPW + b * _ZR, _ZR)])

        plsc.subcore_barrier()

        def prefetch(b, kk):
            jb, eb, ilb, xjb, eab = bufs[b]
            pltpu.async_copy(jr_hbm.at[c, s, pl.ds(kk * _EC, _EC)], jb, sjr[b])
            pltpu.async_copy(er_hbm.at[c, s, pl.ds(kk * _EC, _EC)], eb, ser[b])
            pltpu.async_copy(ir_hbm.at[c, s, pl.ds(kk * _EC, _EC)], ilb, sir[b])

        for b in range(_NBUF):
            prefetch(b, b)

        @pl.loop(0, n2)
        def _iter(k2):
            for b in range(_NBUF):
                jb, eb, ilb, xjb, eab = bufs[b]

                @pl.when(k2 > 0)
                def _drain():
                    pltpu.make_async_copy(xjb, spm.at[ilb], ssc[b]).wait()

                pltpu.make_async_copy(jr_hbm.at[c, s, pl.ds(0, _EC)], jb,
                                      sjr[b]).wait()
                pltpu.async_copy(t_hbm.at[jb], xjb, sg[b])
                pltpu.make_async_copy(er_hbm.at[c, s, pl.ds(0, _EC)], eb,
                                      ser[b]).wait()
                pltpu.async_copy(ea_hbm.at[eb], eab, sge[b])

            for b in range(_NBUF):
                jb, eb, ilb, xjb, eab = bufs[b]
                kk = k2 * _NBUF + b

                pltpu.make_async_copy(ir_hbm.at[c, s, pl.ds(0, _EC)], ilb,
                                      sir[b]).wait()
                pltpu.make_async_copy(t_hbm.at[jb], xjb, sg[b]).wait()
                pltpu.make_async_copy(ea_hbm.at[eb], eab, sge[b]).wait()

                @plsc.parallel_loop(0, _EC, unroll=4)
                def _msg(r):
                    for q in range(HIDDEN // 16):
                        sl = pl.ds(q * 16, 16)
                        v = xjb.at[r, sl][...] + eab.at[r, sl][...]
                        xjb.at[r, sl][...] = jnp.maximum(v, 0.0)

                pltpu.async_copy(xjb, spm.at[ilb], ssc[b], add=True)

                @pl.when(kk + _NBUF < n2 * _NBUF)
                def _next():
                    prefetch(b, kk + _NBUF)

        for b in range(_NBUF):
            jb, eb, ilb, xjb, eab = bufs[b]
            pltpu.make_async_copy(xjb, spm.at[ilb], ssc[b]).wait()

        @pl.when(n80 > n2 * _NBUF)
        def _tail():
            jb, eb, ilb, xjb, eab = bufs[0]
            kk = n80 - 1
            pltpu.sync_copy(jr_hbm.at[c, s, pl.ds(kk * _EC, _EC)], jb)
            pltpu.sync_copy(er_hbm.at[c, s, pl.ds(kk * _EC, _EC)], eb)
            pltpu.sync_copy(ir_hbm.at[c, s, pl.ds(kk * _EC, _EC)], ilb)
            pltpu.sync_copy(t_hbm.at[jb], xjb)
            pltpu.sync_copy(ea_hbm.at[eb], eab)

            @plsc.parallel_loop(0, _EC, unroll=4)
            def _msg_t(r):
                for q in range(HIDDEN // 16):
                    sl = pl.ds(q * 16, 16)
                    v = xjb.at[r, sl][...] + eab.at[r, sl][...]
                    xjb.at[r, sl][...] = jnp.maximum(v, 0.0)

            pltpu.sync_copy(xjb, spm.at[ilb], add=True)

        plsc.subcore_barrier()

        start = jnp.minimum(s * _RPW, _HALF - _RPW)
        pltpu.sync_copy(spm.at[pl.ds(start, _RPW)],
                        out_hbm.at[pl.ds(lo + start, _RPW)])

    return k(t, ea, jr, er, ir, counts)


def _conv(t, ea, route, pmlp, pbn):
    aggr = _edge_aggr(t, ea, *route)
    z, s, ss = _mlp_stats(t, aggr, pmlp)
    return _bn_apply(z, s, ss, pbn)


def kernel(x, edge_index, edge_attr, batch, params):
    p = params
    h = _linear(x, p["node_enc"]["w"], p["node_enc"]["b"], block_rows=2000)
    ea = _linear(edge_attr, p["edge_enc"]["w"], p["edge_enc"]["b"], block_rows=8000)
    src = edge_index[0]
    dst = edge_index[1]

    route_f = _route(src, dst)
    route_b = _route(dst, src)

    xf = _conv(h, ea, route_f, p["f_conv1"], p["f_bn1"])
    xf = _conv(xf, ea, route_f, p["f_conv2"], p["f_bn2"])
    xb = _conv(h, ea, route_b, p["b_conv1"], p["b_bn1"])
    xb = _conv(xb, ea, route_b, p["b_conv2"], p["b_bn2"])

    pooled = _pool(xf, xb, batch)
    return _head(pooled, p["head1"], p["head2"])


# speedup vs baseline: 5.6728x; 1.0009x over previous
import dataclasses
import functools

import jax
import jax.numpy as jnp
from jax import lax
from jax.experimental import pallas as pl
from jax.experimental.pallas import tpu as pltpu
from jax.experimental.pallas import tpu_sc as plsc

N_NODES = 50000
N_EDGES = 800000
HIDDEN = 64
N_GRAPHS = 512

F32 = jnp.float32


def _linear_body(x_ref, w_ref, b_ref, o_ref, *, relu):
    y = jnp.dot(x_ref[...], w_ref[...], preferred_element_type=F32) + b_ref[...]
    if relu:
        y = jnp.maximum(y, 0.0)
    o_ref[...] = y


def _linear(x, w, b, *, relu=False, block_rows):
    m, k = x.shape
    n = w.shape[1]
    assert m % block_rows == 0
    return pl.pallas_call(
        functools.partial(_linear_body, relu=relu),
        grid=(m // block_rows,),
        in_specs=[
            pl.BlockSpec((block_rows, k), lambda i: (i, 0)),
            pl.BlockSpec((k, n), lambda i: (0, 0)),
            pl.BlockSpec((1, n), lambda i: (0, 0)),
        ],
        out_specs=pl.BlockSpec((block_rows, n), lambda i: (i, 0)),
        out_shape=jax.ShapeDtypeStruct((m, n), F32),
    )(x, w, b.reshape(1, n))


def _mlp_stats_body(x_ref, a_ref, w1_ref, b1_ref, w2_ref, b2_ref,
                    z_ref, s_ref, ss_ref):
    h = x_ref[...] + a_ref[...]
    h1 = jnp.dot(h, w1_ref[...], preferred_element_type=F32) + b1_ref[...]
    h1 = jnp.maximum(h1, 0.0)
    z = jnp.dot(h1, w2_ref[...], preferred_element_type=F32) + b2_ref[...]
    z_ref[...] = z

    @pl.when(pl.program_id(0) == 0)
    def _zero():
        s_ref[...] = jnp.zeros_like(s_ref)
        ss_ref[...] = jnp.zeros_like(ss_ref)

    s_ref[...] += jnp.sum(z, axis=0, keepdims=True)
    ss_ref[...] += jnp.sum(z * z, axis=0, keepdims=True)


def _mlp_stats(x, aggr, p, *, block_rows=2000):
    m, d = x.shape
    assert m % block_rows == 0
    w1, b1 = p["l1"]["w"], p["l1"]["b"].reshape(1, d)
    w2, b2 = p["l2"]["w"], p["l2"]["b"].reshape(1, d)
    z, s, ss = pl.pallas_call(
        _mlp_stats_body,
        grid=(m // block_rows,),
        in_specs=[
            pl.BlockSpec((block_rows, d), lambda i: (i, 0)),
            pl.BlockSpec((block_rows, d), lambda i: (i, 0)),
            pl.BlockSpec((d, d), lambda i: (0, 0)),
            pl.BlockSpec((1, d), lambda i: (0, 0)),
            pl.BlockSpec((d, d), lambda i: (0, 0)),
            pl.BlockSpec((1, d), lambda i: (0, 0)),
        ],
        out_specs=[
            pl.BlockSpec((block_rows, d), lambda i: (i, 0)),
            pl.BlockSpec((1, d), lambda i: (0, 0)),
            pl.BlockSpec((1, d), lambda i: (0, 0)),
        ],
        out_shape=[
            jax.ShapeDtypeStruct((m, d), F32),
            jax.ShapeDtypeStruct((1, d), F32),
            jax.ShapeDtypeStruct((1, d), F32),
        ],
    )(x, aggr, w1, b1, w2, b2)
    return z, s, ss


def _bn_apply_body(z_ref, s_ref, ss_ref, g_ref, b_ref, o_ref, *, m):
    mean = s_ref[...] / m
    var = ss_ref[...] / m - mean * mean
    inv = jax.lax.rsqrt(var + 1e-5)
    o_ref[...] = jnp.maximum((z_ref[...] - mean) * inv * g_ref[...] + b_ref[...], 0.0)


def _bn_apply(z, s, ss, p, *, block_rows=2000):
    m, d = z.shape
    return pl.pallas_call(
        functools.partial(_bn_apply_body, m=float(m)),
        grid=(m // block_rows,),
        in_specs=[
            pl.BlockSpec((block_rows, d), lambda i: (i, 0)),
            pl.BlockSpec((1, d), lambda i: (0, 0)),
            pl.BlockSpec((1, d), lambda i: (0, 0)),
            pl.BlockSpec((1, d), lambda i: (0, 0)),
            pl.BlockSpec((1, d), lambda i: (0, 0)),
        ],
        out_specs=pl.BlockSpec((block_rows, d), lambda i: (i, 0)),
        out_shape=jax.ShapeDtypeStruct((m, d), F32),
    )(z, s, ss, p["g"].reshape(1, d), p["b"].reshape(1, d))


def _pool_body(xf_ref, xb_ref, batch_ref, o_ref):
    seg = batch_ref[...].reshape(1, -1).astype(jnp.int32)
    gids = jax.lax.broadcasted_iota(jnp.int32, (N_GRAPHS, seg.shape[1]), 0)
    onehot = (gids == seg).astype(F32)
    h = jnp.concatenate([xf_ref[...], xb_ref[...]], axis=1)

    @pl.when(pl.program_id(0) == 0)
    def _zero():
        o_ref[...] = jnp.zeros_like(o_ref)

    o_ref[...] += jnp.dot(onehot, h, preferred_element_type=F32)


def _pool(xf, xb, batch, *, block_rows=2000):
    m, d = xf.shape
    batch3 = batch.astype(jnp.int32).reshape(m // block_rows, 1, block_rows)
    return pl.pallas_call(
        _pool_body,
        grid=(m // block_rows,),
        in_specs=[
            pl.BlockSpec((block_rows, d), lambda i: (i, 0)),
            pl.BlockSpec((block_rows, d), lambda i: (i, 0)),
            pl.BlockSpec((1, 1, block_rows), lambda i: (i, 0, 0)),
        ],
        out_specs=pl.BlockSpec((N_GRAPHS, 2 * d), lambda i: (0, 0)),
        out_shape=jax.ShapeDtypeStruct((N_GRAPHS, 2 * d), F32),
    )(xf, xb, batch3)


def _head_body(x_ref, w1_ref, b1_ref, w2_ref, b2_ref, o_ref):
    h = jnp.dot(x_ref[...], w1_ref[...], preferred_element_type=F32) + b1_ref[...]
    h = jnp.maximum(h, 0.0)
    o_ref[...] = jnp.dot(h, w2_ref[...], preferred_element_type=F32) + b2_ref[...]


def _head(pooled, p1, p2):
    m, k = pooled.shape
    d = p1["w"].shape[1]
    out = pl.pallas_call(
        _head_body,
        in_specs=[
            pl.BlockSpec((m, k), lambda: (0, 0)),
            pl.BlockSpec((k, d), lambda: (0, 0)),
            pl.BlockSpec((1, d), lambda: (0, 0)),
            pl.BlockSpec((d, 1), lambda: (0, 0)),
            pl.BlockSpec((1, 1), lambda: (0, 0)),
        ],
        out_specs=pl.BlockSpec((m, 1), lambda: (0, 0)),
        out_shape=jax.ShapeDtypeStruct((m, 1), F32),
    )(pooled, p1["w"], p1["b"].reshape(1, d), p2["w"], p2["b"].reshape(1, 1))
    return out[:, 0]


_HALF = N_NODES // 2
_RPW = 1568
_SPM_ROWS = 16 * _RPW
_DUMP = _HALF
_EC = 96
_EPS = N_EDGES // 16
_NCH = _EPS // _EC
_ZR = _RPW // 32


_NBUF = 2

_FB = 960
_FFIN = 1216
_CAPW = 52 * _FB + _FFIN
_CBUF = 1280
_RCH = 10000


def _route(jarr, iarr):
    mesh = plsc.VectorSubcoreMesh(core_axis_name="c", subcore_axis_name="s")

    out_type = [
        jax.ShapeDtypeStruct((2, 16, _CAPW), jnp.int32),
        jax.ShapeDtypeStruct((2, 16, _CAPW), jnp.int32),
        jax.ShapeDtypeStruct((2, 16, _CAPW), jnp.int32),
        jax.ShapeDtypeStruct((2, 16, 16), jnp.int32),
    ]
    scratch = [
        pltpu.VMEM((_RCH,), jnp.int32),
        pltpu.VMEM((_RCH,), jnp.int32),
        pltpu.VMEM((_CBUF,), jnp.int32),
        pltpu.VMEM((_CBUF,), jnp.int32),
        pltpu.VMEM((_CBUF,), jnp.int32),
        pltpu.VMEM((16,), jnp.int32),
    ]

    cp = pltpu.CompilerParams(use_tc_tiling_on_sc=False)
    if "needs_layout_passes" in pltpu.CompilerParams.__dataclass_fields__:
        cp = dataclasses.replace(cp, needs_layout_passes=False)

    @functools.partial(
        pl.kernel,
        mesh=mesh,
        out_type=out_type,
        compiler_params=cp,
        scratch_types=scratch,
    )
    def k(j_hbm, i_hbm, jr, er, ir, counts, jc, ic, bj, be, bi, cntb):
        c = lax.axis_index("c")
        s = lax.axis_index("s")
        lo = c * _HALF
        lanes = lax.iota(jnp.int32, 16)

        def group(g, carry, chunk_base):
            cur, nblk = carry
            off = g * 16
            iv = ic.at[pl.ds(off, 16)][...]
            jv = jc.at[pl.ds(off, 16)][...]
            ev = chunk_base + off + lanes
            ilv = iv - lo
            m = (ilv >= 0) & (ilv < _HALF)
            plsc.store_compressed(bj.at[pl.ds(cur, 16)], jv, mask=m)
            plsc.store_compressed(be.at[pl.ds(cur, 16)], ev, mask=m)
            plsc.store_compressed(bi.at[pl.ds(cur, 16)], ilv, mask=m)
            cnt = jnp.max(plsc.all_reduce_population_count(m))
            cur = cur + cnt
            flush = cur >= _FB

            @pl.when(flush)
            def _flush():
                pltpu.sync_copy(bj.at[pl.ds(0, _FB)],
                                jr.at[c, s, pl.ds(nblk * _FB, _FB)])
                pltpu.sync_copy(be.at[pl.ds(0, _FB)],
                                er.at[c, s, pl.ds(nblk * _FB, _FB)])
                pltpu.sync_copy(bi.at[pl.ds(0, _FB)],
                                ir.at[c, s, pl.ds(nblk * _FB, _FB)])
                bj.at[pl.ds(0, 16)][...] = bj.at[pl.ds(_FB, 16)][...]
                be.at[pl.ds(0, 16)][...] = be.at[pl.ds(_FB, 16)][...]
                bi.at[pl.ds(0, 16)][...] = bi.at[pl.ds(_FB, 16)][...]

            cur = jnp.where(flush, cur - _FB, cur)
            nblk = nblk + flush.astype(jnp.int32)
            return cur, nblk

        carry = (jnp.int32(0), jnp.int32(0))
        for ch in range(_EPS // _RCH):
            base = s * _EPS + ch * _RCH
            pltpu.sync_copy(j_hbm.at[pl.ds(base, _RCH)], jc)
            pltpu.sync_copy(i_hbm.at[pl.ds(base, _RCH)], ic)
            carry = lax.fori_loop(
                0, _RCH // 16,
                functools.partial(group, chunk_base=base), carry)
        cur, nblk = carry

        zv16 = jnp.zeros((16,), jnp.int32)
        dumpv = _DUMP + lanes

        @plsc.parallel_loop(0, 2 * _EC, step=16)
        def _pad(kq):
            bj.at[pl.ds(cur + kq, 16)][...] = zv16
            be.at[pl.ds(cur + kq, 16)][...] = zv16
            bi.at[pl.ds(cur + kq, 16)][...] = dumpv

        cur_pad = jnp.maximum(((cur + _EC - 1) // _EC) * _EC, 2 * _EC)
        pltpu.sync_copy(bj.at[pl.ds(0, _FFIN)],
                        jr.at[c, s, pl.ds(nblk * _FB, _FFIN)])
        pltpu.sync_copy(be.at[pl.ds(0, _FFIN)],
                        er.at[c, s, pl.ds(nblk * _FB, _FFIN)])
        pltpu.sync_copy(bi.at[pl.ds(0, _FFIN)],
                        ir.at[c, s, pl.ds(nblk * _FB, _FFIN)])

        total = nblk * _FB + cur_pad
        cntb[...] = jnp.where(lanes == 0, total, 0)
        pltpu.sync_copy(cntb, counts.at[c, s])

    return k(jarr, iarr)


def _edge_aggr(t, ea, jr, er, ir, counts):
    mesh = plsc.VectorSubcoreMesh(core_axis_name="c", subcore_axis_name="s")

    scratch = []
    for _ in range(_NBUF):
        scratch += [
            pltpu.VMEM((_EC,), jnp.int32),
            pltpu.VMEM((_EC,), jnp.int32),
            pltpu.VMEM((_EC,), jnp.int32),
            pltpu.VMEM((_EC, HIDDEN), F32),
            pltpu.VMEM((_EC, HIDDEN), F32),
        ]
    scratch += [
        pltpu.VMEM((_ZR, HIDDEN), F32),
        pltpu.VMEM_SHARED((_SPM_ROWS, HIDDEN), F32),
        pltpu.VMEM((16,), jnp.int32),
    ]
    scratch += [pltpu.SemaphoreType.DMA] * (6 * _NBUF)

    cp = pltpu.CompilerParams(use_tc_tiling_on_sc=False)
    if "needs_layout_passes" in pltpu.CompilerParams.__dataclass_fields__:
        cp = dataclasses.replace(cp, needs_layout_passes=False)

    @functools.partial(
        pl.kernel,
        mesh=mesh,
        out_type=jax.ShapeDtypeStruct((N_NODES, HIDDEN), F32),
        compiler_params=cp,
        scratch_types=scratch,
    )
    def k(t_hbm, ea_hbm, jr_hbm, er_hbm, ir_hbm, cnt_hbm, out_hbm, *refs):
        bufs = [refs[5 * b:5 * b + 5] for b in range(_NBUF)]
        zbuf = refs[5 * _NBUF]
        spm = refs[5 * _NBUF + 1]
        cntb = refs[5 * _NBUF + 2]
        sems = refs[5 * _NBUF + 3:]
        sjr = sems[0:_NBUF]
        ser = sems[_NBUF:2 * _NBUF]
        sir = sems[2 * _NBUF:3 * _NBUF]
        sg = sems[3 * _NBUF:4 * _NBUF]
        sge = sems[4 * _NBUF:5 * _NBUF]
        ssc = sems[5 * _NBUF:6 * _NBUF]

        c = lax.axis_index("c")
        s = lax.axis_index("s")
        lo = c * _HALF

        pltpu.sync_copy(cnt_hbm.at[c, s], cntb)
        n80 = jnp.max(cntb[...]) // _EC
        n2 = n80 // _NBUF

        zv = jnp.zeros((16,), F32)

        @pl.loop(0, _ZR)
        def _zrow(r):
            for q in range(HIDDEN // 16):
                zbuf.at[r, pl.ds(q * 16, 16)][...] = zv

        @pl.loop(0, 32)
        def _zcopy(b):
            pltpu.sync_copy(zbuf, spm.at[pl.ds(s * _RPW + b * _ZR, _ZR)])

        plsc.subcore_barrier()

        def prefetch(b, kk):
            jb, eb, ilb, xjb, eab = bufs[b]
            pltpu.async_copy(jr_hbm.at[c, s, pl.ds(kk * _EC, _EC)], jb, sjr[b])
            pltpu.async_copy(er_hbm.at[c, s, pl.ds(kk * _EC, _EC)], eb, ser[b])
            pltpu.async_copy(ir_hbm.at[c, s, pl.ds(kk * _EC, _EC)], ilb, sir[b])

        for b in range(_NBUF):
            prefetch(b, b)

        @pl.loop(0, n2)
        def _iter(k2):
            for b in range(_NBUF):
                jb, eb, ilb, xjb, eab = bufs[b]

                @pl.when(k2 > 0)
                def _drain():
                    pltpu.make_async_copy(xjb, spm.at[ilb], ssc[b]).wait()

                pltpu.make_async_copy(jr_hbm.at[c, s, pl.ds(0, _EC)], jb,
                                      sjr[b]).wait()
                pltpu.async_copy(t_hbm.at[jb], xjb, sg[b])
                pltpu.make_async_copy(er_hbm.at[c, s, pl.ds(0, _EC)], eb,
                                      ser[b]).wait()
                pltpu.async_copy(ea_hbm.at[eb], eab, sge[b])

            for b in range(_NBUF):
                jb, eb, ilb, xjb, eab = bufs[b]
                kk = k2 * _NBUF + b

                pltpu.make_async_copy(ir_hbm.at[c, s, pl.ds(0, _EC)], ilb,
                                      sir[b]).wait()
                pltpu.make_async_copy(t_hbm.at[jb], xjb, sg[b]).wait()
                pltpu.make_async_copy(ea_hbm.at[eb], eab, sge[b]).wait()

                @plsc.parallel_loop(0, _EC, unroll=4)
                def _msg(r):
                    for q in range(HIDDEN // 16):
                        sl = pl.ds(q * 16, 16)
                        v = xjb.at[r, sl][...] + eab.at[r, sl][...]
                        xjb.at[r, sl][...] = jnp.maximum(v, 0.0)

                pltpu.async_copy(xjb, spm.at[ilb], ssc[b], add=True)

                @pl.when(kk + _NBUF < n2 * _NBUF)
                def _next():
                    prefetch(b, kk + _NBUF)

        for b in range(_NBUF):
            jb, eb, ilb, xjb, eab = bufs[b]
            pltpu.make_async_copy(xjb, spm.at[ilb], ssc[b]).wait()

        @pl.when(n80 > n2 * _NBUF)
        def _tail():
            jb, eb, ilb, xjb, eab = bufs[0]
            kk = n80 - 1
            pltpu.sync_copy(jr_hbm.at[c, s, pl.ds(kk * _EC, _EC)], jb)
            pltpu.sync_copy(er_hbm.at[c, s, pl.ds(kk * _EC, _EC)], eb)
            pltpu.sync_copy(ir_hbm.at[c, s, pl.ds(kk * _EC, _EC)], ilb)
            pltpu.sync_copy(t_hbm.at[jb], xjb)
            pltpu.sync_copy(ea_hbm.at[eb], eab)

            @plsc.parallel_loop(0, _EC, unroll=4)
            def _msg_t(r):
                for q in range(HIDDEN // 16):
                    sl = pl.ds(q * 16, 16)
                    v = xjb.at[r, sl][...] + eab.at[r, sl][...]
                    xjb.at[r, sl][...] = jnp.maximum(v, 0.0)

            pltpu.sync_copy(xjb, spm.at[ilb], add=True)

        plsc.subcore_barrier()

        start = jnp.minimum(s * _RPW, _HALF - _RPW)
        pltpu.sync_copy(spm.at[pl.ds(start, _RPW)],
                        out_hbm.at[pl.ds(lo + start, _RPW)])

    return k(t, ea, jr, er, ir, counts)


def _conv(t, ea, route, pmlp, pbn):
    aggr = _edge_aggr(t, ea, *route)
    z, s, ss = _mlp_stats(t, aggr, pmlp)
    return _bn_apply(z, s, ss, pbn)


def kernel(x, edge_index, edge_attr, batch, params):
    p = params
    h = _linear(x, p["node_enc"]["w"], p["node_enc"]["b"], block_rows=2000)
    ea = _linear(edge_attr, p["edge_enc"]["w"], p["edge_enc"]["b"], block_rows=8000)
    src = edge_index[0]
    dst = edge_index[1]

    route_f = _route(src, dst)
    route_b = _route(dst, src)

    xf = _conv(h, ea, route_f, p["f_conv1"], p["f_bn1"])
    xb = _conv(h, ea, route_b, p["b_conv1"], p["b_bn1"])
    xf = _conv(xf, ea, route_f, p["f_conv2"], p["f_bn2"])
    xb = _conv(xb, ea, route_b, p["b_conv2"], p["b_bn2"])

    pooled = _pool(xf, xb, batch)
    return _head(pooled, p["head1"], p["head2"])
